# baseline (speedup 1.0000x reference)
import dataclasses
import functools

import jax
import jax.numpy as jnp
from jax import lax
from jax.experimental import pallas as pl
from jax.experimental.pallas import tpu as pltpu
from jax.experimental.pallas import tpu_sc as plsc

_NC = 2
_NS = 16
_NW = _NC * _NS
_L = 16
_B = 128

_F32 = jnp.float32
_I32 = jnp.int32


def _sc_compiler_params():
    cp = pltpu.CompilerParams()
    if "needs_layout_passes" in pltpu.CompilerParams.__dataclass_fields__:
        cp = dataclasses.replace(cp, needs_layout_passes=False)
    return cp


def _vector_mesh():
    return plsc.VectorSubcoreMesh(core_axis_name="c", subcore_axis_name="s",
                                  num_cores=_NC, num_subcores=_NS)


def _make_deg_kernel(n_acc, n_rng, rng):
    rpt = rng // _NS

    @functools.partial(
        pl.kernel,
        out_type=jax.ShapeDtypeStruct((_NC, n_acc, 128), _F32),
        mesh=_vector_mesh(),
        scratch_types=[
            pltpu.VMEM((16,), _I32),
            pltpu.VMEM((_B,), _I32),
            pltpu.VMEM((_B, 128), _F32),
            pltpu.VMEM((rpt, 128), _F32),
            pltpu.VMEM_SHARED((rng, 128), _F32),
        ],
    )
    def kd(coli_hbm, goff_hbm, out_hbm, offs_v, ci_v, ones_v, zbuf, acc_sh):
        cid = lax.axis_index("c")
        sid = lax.axis_index("s")
        wid = sid * _NC + cid
        zero16 = jnp.zeros((_L,), _F32)
        e0 = jnp.where(lax.iota(_I32, _L) == 0, 1.0, 0.0).astype(_F32)

        pltpu.sync_copy(goff_hbm, offs_v)
        offs = offs_v[...]

        @pl.loop(0, rpt)
        def _(i):
            for kk in range(128 // _L):
                zbuf[i, pl.ds(kk * _L, _L)] = zero16

        @pl.loop(0, _B)
        def _(i):
            ones_v[i, pl.ds(0, _L)] = e0
            for kk in range(1, 128 // _L):
                ones_v[i, pl.ds(kk * _L, _L)] = zero16

        for r in range(n_rng):
            pltpu.sync_copy(zbuf, acc_sh.at[pl.ds(sid * rpt, rpt)])
            plsc.subcore_barrier()
            b0 = offs[r] // _B
            b1 = offs[r + 1] // _B

            @pl.loop(b0 + wid, b1, step=_NW)
            def _(b):
                pltpu.sync_copy(coli_hbm.at[pl.ds(b * _B, _B)], ci_v)
                pltpu.sync_copy(ones_v, acc_sh.at[ci_v], add=True)

            plsc.subcore_barrier()
            pltpu.sync_copy(
                acc_sh.at[pl.ds(sid * rpt, rpt)],
                out_hbm.at[cid, pl.ds(r * rng + sid * rpt, rpt)])
            plsc.subcore_barrier()

    return kd


def _make_nsum_kernel(n_acc, width, n_rng, rng):
    rpt = rng // _NS

    @functools.partial(
        pl.kernel,
        out_type=jax.ShapeDtypeStruct((_NC, n_acc, width), _F32),
        mesh=_vector_mesh(),
        scratch_types=[
            pltpu.VMEM((16,), _I32),
            pltpu.VMEM((_B,), _I32),
            pltpu.VMEM((_B,), _I32),
            pltpu.VMEM((_B, width), _F32),
            pltpu.VMEM((rpt, width), _F32),
            pltpu.VMEM_SHARED((rng, width), _F32),
        ],
    )
    def ka(tbl_hbm, rowi_hbm, coli_hbm, goff_hbm, out_hbm,
           offs_v, ri_v, ci_v, rows_v, zbuf, acc_sh):
        cid = lax.axis_index("c")
        sid = lax.axis_index("s")
        wid = sid * _NC + cid
        zero16 = jnp.zeros((_L,), _F32)

        pltpu.sync_copy(goff_hbm, offs_v)
        offs = offs_v[...]

        @pl.loop(0, rpt)
        def _(i):
            for kk in range(width // _L):
                zbuf[i, pl.ds(kk * _L, _L)] = zero16

        for r in range(n_rng):
            pltpu.sync_copy(zbuf, acc_sh.at[pl.ds(sid * rpt, rpt)])
            plsc.subcore_barrier()
            b0 = offs[r] // _B
            b1 = offs[r + 1] // _B

            @pl.loop(b0 + wid, b1, step=_NW)
            def _(b):
                off = b * _B
                pltpu.sync_copy(rowi_hbm.at[pl.ds(off, _B)], ri_v)
                pltpu.sync_copy(coli_hbm.at[pl.ds(off, _B)], ci_v)
                pltpu.sync_copy(tbl_hbm.at[ri_v], rows_v)
                pltpu.sync_copy(rows_v, acc_sh.at[ci_v], add=True)

            plsc.subcore_barrier()
            pltpu.sync_copy(
                acc_sh.at[pl.ds(sid * rpt, rpt)],
                out_hbm.at[cid, pl.ds(r * rng + sid * rpt, rpt)])
            plsc.subcore_barrier()

    return ka


def _make_edge_kernel(n_acc, dout, rw, n_rng, rng):
    rpt = rng // _NS
    dp = 128

    @functools.partial(
        pl.kernel,
        out_type=jax.ShapeDtypeStruct((_NC, n_acc, dp), _F32),
        mesh=_vector_mesh(),
        scratch_types=[
            pltpu.VMEM((16,), _I32),
            pltpu.VMEM((_B,), _I32),
            pltpu.VMEM((_B,), _I32),
            pltpu.VMEM((_B,), _I32),
            pltpu.VMEM((_B, 384), _F32),
            pltpu.VMEM((_B, rw), _F32),
            pltpu.VMEM((_B, dp), _F32),
            pltpu.VMEM_SHARED((rng, dp), _F32),
        ],
        compiler_params=_sc_compiler_params(),
    )
    def kb(c_hbm, r_hbm, rowi_hbm, colg_hbm, coll_hbm, goff_hbm, z_hbm,
           out_hbm, offs_v, ri_v, cg_v, cl_v, crows, rrows, obuf, acc_sh):
        cid = lax.axis_index("c")
        sid = lax.axis_index("s")
        wid = sid * _NC + cid
        iota16 = lax.iota(_I32, _L)

        pltpu.sync_copy(goff_hbm, offs_v)
        offs = offs_v[...]
        pltpu.sync_copy(z_hbm, obuf)

        for r in range(n_rng):
            pltpu.sync_copy(z_hbm.at[pl.ds(0, rpt)],
                            acc_sh.at[pl.ds(sid * rpt, rpt)])
            plsc.subcore_barrier()
            b0 = offs[r] // _B
            b1 = offs[r + 1] // _B

            @pl.loop(b0 + wid, b1, step=_NW)
            def _(b):
                off = b * _B
                pltpu.sync_copy(rowi_hbm.at[pl.ds(off, _B)], ri_v)
                pltpu.sync_copy(colg_hbm.at[pl.ds(off, _B)], cg_v)
                pltpu.sync_copy(coll_hbm.at[pl.ds(off, _B)], cl_v)
                pltpu.sync_copy(c_hbm.at[cg_v], crows)
                pltpu.sync_copy(r_hbm.at[ri_v], rrows)

                @pl.loop(0, _B // _L)
                def _(g16):
                    eidx = iota16 + g16 * _L

                    def dbody(d, accs):
                        ss, tn, wi, ni, nj = accs
                        dv = jnp.full((_L,), d, _I32)
                        xc = plsc.load_gather(crows, [eidx, dv])
                        tt = plsc.load_gather(crows, [eidx, dv + 128])
                        gg = plsc.load_gather(crows, [eidx, dv + 256])
                        xj = plsc.load_gather(rrows, [eidx, dv])
                        df = xj - xc
                        dd = df * df
                        u = xc * xj
                        ss = ss + dd
                        tn = tn + tt * dd
                        wi = wi + gg * u
                        ni = ni + gg * (xc * xc)
                        nj = nj + gg * (xj * xj)
                        return (ss, tn, wi, ni, nj)

                    z = jnp.zeros((_L,), _F32)
                    ss, tn, wi, ni, nj = lax.fori_loop(
                        0, 128, dbody, (z, z, z, z, z), unroll=4)
                    tau = tn / jnp.maximum(ss, 1e-16)
                    p = jnp.maximum(ni * nj, 1e-30)
                    bits = plsc.bitcast(p, _I32)
                    zb = plsc.bitcast(
                        jnp.int32(0x5F3759DF) - (bits >> 1), _F32)
                    for _ in range(3):
                        zb = zb * (1.5 - 0.5 * p * zb * zb)
                    den = p * zb + 1e-8
                    arg = wi / den
                    alpha = 1.0 / (1.0 + jnp.exp(-arg))
                    cvec = tau * alpha

                    def xmbody(d, carry):
                        dv = jnp.full((_L,), d, _I32)
                        xmv = plsc.load_gather(rrows, [eidx, dv + 128])
                        plsc.store_scatter(obuf, [eidx, dv], xmv * cvec)
                        return carry

                    lax.fori_loop(0, dout, xmbody, 0, unroll=4)

                pltpu.sync_copy(obuf, acc_sh.at[cl_v], add=True)

            plsc.subcore_barrier()
            pltpu.sync_copy(
                acc_sh.at[pl.ds(sid * rpt, rpt)],
                out_hbm.at[cid, pl.ds(r * rng + sid * rpt, rpt)])
            plsc.subcore_barrier()

    return kb


def _ln(h, g, b):
    mu = jnp.mean(h, axis=-1, keepdims=True)
    var = jnp.mean((h - mu) * (h - mu), axis=-1, keepdims=True)
    return (h - mu) / jnp.sqrt(var + 1e-5) * g + b


_BLK = 1024


def _tc_encoder(xp, p):
    n_pad = xp.shape[0]

    def body(x_ref, w_ref, b_ref, g_ref, bt_ref, o_ref):
        h = jnp.dot(x_ref[...], w_ref[...],
                    preferred_element_type=_F32) + b_ref[...]
        o_ref[...] = jnp.maximum(_ln(h, g_ref[...], bt_ref[...]), 0.0)

    d_in = xp.shape[1]
    hid = p['enc_W'].shape[1]
    return pl.pallas_call(
        body,
        grid=(n_pad // _BLK,),
        in_specs=[
            pl.BlockSpec((_BLK, d_in), lambda i: (i, 0)),
            pl.BlockSpec((d_in, hid), lambda i: (0, 0)),
            pl.BlockSpec((hid,), lambda i: (0,)),
            pl.BlockSpec((hid,), lambda i: (0,)),
            pl.BlockSpec((hid,), lambda i: (0,)),
        ],
        out_specs=pl.BlockSpec((_BLK, hid), lambda i: (i, 0)),
        out_shape=jax.ShapeDtypeStruct((n_pad, hid), _F32),
    )(xp, p['enc_W'], p['enc_b'], p['enc_g'], p['enc_bt'])


def _tc_dense(h, ns2, deg2, l, dout):
    n_pad = h.shape[0]
    rw = 256

    def body(h_ref, ns_ref, deg_ref, mw1_ref, mb1_ref, mg_ref, mbt_ref,
             mw2_ref, mb2_ref, msgw_ref, selfw_ref, selfb_ref,
             c_ref, r_ref, so_ref):
        hh = h_ref[...]
        s = ns_ref[0] + ns_ref[1] + hh
        cnt = deg_ref[...] + 1.0
        x_nb = s / jnp.maximum(cnt, 1.0)[:, None]
        comb = jnp.concatenate([hh, x_nb], axis=-1)
        hm = jnp.dot(comb, mw1_ref[...],
                     preferred_element_type=_F32) + mb1_ref[...]
        hm = jnp.maximum(_ln(hm, mg_ref[...], mbt_ref[...]), 0.0)
        raw = jnp.dot(hm, mw2_ref[...],
                      preferred_element_type=_F32) + mb2_ref[...]
        z2 = 2.0 * raw
        sp = jnp.maximum(z2, 0.0) + jnp.log1p(jnp.exp(-jnp.abs(z2)))
        g = jnp.clip(sp * 0.5, 0.001, 10.0)
        t = (1.0 - g * g) / (1.0 + g * g)
        xm = jnp.dot(hh, msgw_ref[...], preferred_element_type=_F32)
        so = jnp.dot(hh, selfw_ref[...],
                     preferred_element_type=_F32) + selfb_ref[...]
        c_ref[...] = jnp.concatenate([hh, t, g], axis=-1)
        if dout == 128:
            r_ref[...] = jnp.concatenate([hh, xm], axis=-1)
        else:
            pad = jnp.zeros((_BLK, 128 - dout), _F32)
            r_ref[...] = jnp.concatenate([hh, xm, pad], axis=-1)
        so_ref[...] = so

    return pl.pallas_call(
        body,
        grid=(n_pad // _BLK,),
        in_specs=[
            pl.BlockSpec((_BLK, 128), lambda i: (i, 0)),
            pl.BlockSpec((2, _BLK, 128), lambda i: (0, i, 0)),
            pl.BlockSpec((_BLK,), lambda i: (i,)),
            pl.BlockSpec((256, 64), lambda i: (0, 0)),
            pl.BlockSpec((64,), lambda i: (0,)),
            pl.BlockSpec((64,), lambda i: (0,)),
            pl.BlockSpec((64,), lambda i: (0,)),
            pl.BlockSpec((64, 128), lambda i: (0, 0)),
            pl.BlockSpec((128,), lambda i: (0,)),
            pl.BlockSpec((128, dout), lambda i: (0, 0)),
            pl.BlockSpec((128, dout), lambda i: (0, 0)),
            pl.BlockSpec((dout,), lambda i: (0,)),
        ],
        out_specs=[
            pl.BlockSpec((_BLK, 384), lambda i: (i, 0)),
            pl.BlockSpec((_BLK, rw), lambda i: (i, 0)),
            pl.BlockSpec((_BLK, dout), lambda i: (i, 0)),
        ],
        out_shape=[
            jax.ShapeDtypeStruct((n_pad, 384), _F32),
            jax.ShapeDtypeStruct((n_pad, rw), _F32),
            jax.ShapeDtypeStruct((n_pad, dout), _F32),
        ],
    )(h, ns2, deg2, l['mW1'], l['mb1'], l['mg'], l['mbt'],
      l['mW2'], l['mb2'], l['msgW'], l['selfW'], l['selfb'])


def _tc_epilogue(agg2, so, l, last):
    n_pad = so.shape[0]
    dout = so.shape[1]

    def body(agg_ref, so_ref, g_ref, bt_ref, o_ref):
        agg = agg_ref[0][:, :dout] + agg_ref[1][:, :dout]
        o = _ln(agg + so_ref[...], g_ref[...], bt_ref[...])
        if last:
            m = jnp.max(o, axis=-1, keepdims=True)
            lse = jnp.log(jnp.sum(jnp.exp(o - m), axis=-1, keepdims=True)) + m
            o_ref[...] = o - lse
        else:
            o_ref[...] = jnp.maximum(o, 0.0)

    return pl.pallas_call(
        body,
        grid=(n_pad // _BLK,),
        in_specs=[
            pl.BlockSpec((2, _BLK, 128), lambda i: (0, i, 0)),
            pl.BlockSpec((_BLK, dout), lambda i: (i, 0)),
            pl.BlockSpec((dout,), lambda i: (0,)),
            pl.BlockSpec((dout,), lambda i: (0,)),
        ],
        out_specs=pl.BlockSpec((_BLK, dout), lambda i: (i, 0)),
        out_shape=jax.ShapeDtypeStruct((n_pad, dout), _F32),
    )(agg2, so, l['ng'], l['nbt'])


def kernel(x, params, edge_index):
    n, d_in = x.shape
    e = edge_index.shape[1]

    n_rng = 8
    rng = _NS * 80
    n_acc = n_rng * rng
    row = edge_index[0]
    col = edge_index[1]

    grp = col // rng
    pos = jnp.zeros((e,), _I32)
    goffs = []
    base = jnp.int32(0)
    pad_counts = []
    for r in range(n_rng):
        m = grp == r
        ranks = jnp.cumsum(m.astype(_I32))
        size = ranks[-1]
        goffs.append(base)
        pos = jnp.where(m, base + ranks - 1, pos)
        padded = ((size + _B - 1) // _B) * _B
        pad_counts.append(padded - size)
        base = base + padded
    goffs.append(base)
    ep = e + n_rng * _B
    rowp = jnp.full((ep,), n, _I32).at[pos].set(row)
    colp = jnp.zeros((ep,), _I32).at[pos].set(col - grp * rng)
    colg = jnp.full((ep,), n, _I32).at[pos].set(col)
    goff = jnp.zeros((16,), _I32).at[:n_rng + 1].set(jnp.stack(goffs))
    padvec = jnp.zeros((n_acc,), _F32).at[
        jnp.arange(n_rng) * rng].set(jnp.stack(pad_counts).astype(_F32))

    xp = jnp.pad(x, ((0, n_acc - n), (0, 0)))
    zblk = jnp.zeros((_B, 128), _F32)

    deg_out = _make_deg_kernel(n_acc, n_rng, rng)(colp, goff)
    deg = deg_out[0, :, 0] + deg_out[1, :, 0] - padvec
    h = _tc_encoder(xp, params)

    nsum = _make_nsum_kernel(n_acc, 128, n_rng, rng)
    n_layers = len(params['layers'])
    for i, l in enumerate(params['layers']):
        dout = l['msgW'].shape[1]
        ns2 = nsum(h, rowp, colp, goff)
        c_tbl, r_tbl, so = _tc_dense(h, ns2, deg, l, dout)
        agg2 = _make_edge_kernel(n_acc, dout, 256, n_rng, rng)(
            c_tbl, r_tbl, rowp, colg, colp, goff, zblk)
        h = _tc_epilogue(agg2, so, l, last=(i == n_layers - 1))

    return h[:n]


# speedup vs baseline: 1.7709x; 1.7709x over previous
import dataclasses
import functools

import jax
import jax.numpy as jnp
from jax import lax
from jax.experimental import pallas as pl
from jax.experimental.pallas import tpu as pltpu
from jax.experimental.pallas import tpu_sc as plsc

_NC = 2
_NS = 16
_NW = _NC * _NS
_L = 16
_B = 128

_F32 = jnp.float32
_I32 = jnp.int32


def _sc_compiler_params():
    cp = pltpu.CompilerParams()
    if "needs_layout_passes" in pltpu.CompilerParams.__dataclass_fields__:
        cp = dataclasses.replace(cp, needs_layout_passes=False)
    return cp


def _vector_mesh():
    return plsc.VectorSubcoreMesh(core_axis_name="c", subcore_axis_name="s",
                                  num_cores=_NC, num_subcores=_NS)


def _make_deg_kernel(n_acc, n_rng, rng):
    rpt = rng // _NS

    @functools.partial(
        pl.kernel,
        out_type=jax.ShapeDtypeStruct((_NC, n_acc, 128), _F32),
        mesh=_vector_mesh(),
        scratch_types=[
            pltpu.VMEM((16,), _I32),
            pltpu.VMEM((_B,), _I32),
            pltpu.VMEM((_B, 128), _F32),
            pltpu.VMEM((rpt, 128), _F32),
            pltpu.VMEM_SHARED((rng, 128), _F32),
        ],
    )
    def kd(coli_hbm, goff_hbm, out_hbm, offs_v, ci_v, ones_v, zbuf, acc_sh):
        cid = lax.axis_index("c")
        sid = lax.axis_index("s")
        wid = sid * _NC + cid
        zero16 = jnp.zeros((_L,), _F32)
        e0 = jnp.where(lax.iota(_I32, _L) == 0, 1.0, 0.0).astype(_F32)

        pltpu.sync_copy(goff_hbm, offs_v)
        offs = offs_v[...]

        @pl.loop(0, rpt)
        def _(i):
            for kk in range(128 // _L):
                zbuf[i, pl.ds(kk * _L, _L)] = zero16

        @pl.loop(0, _B)
        def _(i):
            ones_v[i, pl.ds(0, _L)] = e0
            for kk in range(1, 128 // _L):
                ones_v[i, pl.ds(kk * _L, _L)] = zero16

        for r in range(n_rng):
            pltpu.sync_copy(zbuf, acc_sh.at[pl.ds(sid * rpt, rpt)])
            plsc.subcore_barrier()
            b0 = offs[r] // _B
            b1 = offs[r + 1] // _B

            @pl.loop(b0 + wid, b1, step=_NW)
            def _(b):
                pltpu.sync_copy(coli_hbm.at[pl.ds(b * _B, _B)], ci_v)
                pltpu.sync_copy(ones_v, acc_sh.at[ci_v], add=True)

            plsc.subcore_barrier()
            pltpu.sync_copy(
                acc_sh.at[pl.ds(sid * rpt, rpt)],
                out_hbm.at[cid, pl.ds(r * rng + sid * rpt, rpt)])
            plsc.subcore_barrier()

    return kd


def _make_nsum_kernel(n_acc, width, n_rng, rng):
    rpt = rng // _NS

    @functools.partial(
        pl.kernel,
        out_type=jax.ShapeDtypeStruct((_NC, n_acc, width), _F32),
        mesh=_vector_mesh(),
        scratch_types=[
            pltpu.VMEM((16,), _I32),
            pltpu.VMEM((_B,), _I32),
            pltpu.VMEM((_B,), _I32),
            pltpu.VMEM((_B, width), _F32),
            pltpu.VMEM((rpt, width), _F32),
            pltpu.VMEM_SHARED((rng, width), _F32),
        ],
    )
    def ka(tbl_hbm, rowi_hbm, coli_hbm, goff_hbm, out_hbm,
           offs_v, ri_v, ci_v, rows_v, zbuf, acc_sh):
        cid = lax.axis_index("c")
        sid = lax.axis_index("s")
        wid = sid * _NC + cid
        zero16 = jnp.zeros((_L,), _F32)

        pltpu.sync_copy(goff_hbm, offs_v)
        offs = offs_v[...]

        @pl.loop(0, rpt)
        def _(i):
            for kk in range(width // _L):
                zbuf[i, pl.ds(kk * _L, _L)] = zero16

        for r in range(n_rng):
            pltpu.sync_copy(zbuf, acc_sh.at[pl.ds(sid * rpt, rpt)])
            plsc.subcore_barrier()
            b0 = offs[r] // _B
            b1 = offs[r + 1] // _B

            @pl.loop(b0 + wid, b1, step=_NW)
            def _(b):
                off = b * _B
                pltpu.sync_copy(rowi_hbm.at[pl.ds(off, _B)], ri_v)
                pltpu.sync_copy(coli_hbm.at[pl.ds(off, _B)], ci_v)
                pltpu.sync_copy(tbl_hbm.at[ri_v], rows_v)
                pltpu.sync_copy(rows_v, acc_sh.at[ci_v], add=True)

            plsc.subcore_barrier()
            pltpu.sync_copy(
                acc_sh.at[pl.ds(sid * rpt, rpt)],
                out_hbm.at[cid, pl.ds(r * rng + sid * rpt, rpt)])
            plsc.subcore_barrier()

    return ka


def _make_gather_kernel(ep, blocks_per_tile):
    @functools.partial(
        pl.kernel,
        out_type=[
            jax.ShapeDtypeStruct((ep, 384), _F32),
            jax.ShapeDtypeStruct((ep, 256), _F32),
        ],
        mesh=_vector_mesh(),
        scratch_types=[
            pltpu.VMEM((_B,), _I32),
            pltpu.VMEM((_B,), _I32),
            pltpu.VMEM((_B, 384), _F32),
            pltpu.VMEM((_B, 256), _F32),
        ],
    )
    def kg(c_hbm, r_hbm, rowi_hbm, colg_hbm, ec_hbm, er_hbm,
           ri_v, cg_v, crows, rrows):
        cid = lax.axis_index("c")
        sid = lax.axis_index("s")
        wid = sid * _NC + cid
        ebase = wid * blocks_per_tile * _B

        @pl.loop(0, blocks_per_tile)
        def _(b):
            off = ebase + b * _B
            pltpu.sync_copy(rowi_hbm.at[pl.ds(off, _B)], ri_v)
            pltpu.sync_copy(colg_hbm.at[pl.ds(off, _B)], cg_v)
            pltpu.sync_copy(c_hbm.at[cg_v], crows)
            pltpu.sync_copy(r_hbm.at[ri_v], rrows)
            pltpu.sync_copy(crows, ec_hbm.at[pl.ds(off, _B)])
            pltpu.sync_copy(rrows, er_hbm.at[pl.ds(off, _B)])

    return kg


def _make_scatter_kernel(n_acc, n_rng, rng):
    rpt = rng // _NS
    dp = 128

    @functools.partial(
        pl.kernel,
        out_type=jax.ShapeDtypeStruct((_NC, n_acc, dp), _F32),
        mesh=_vector_mesh(),
        scratch_types=[
            pltpu.VMEM((16,), _I32),
            pltpu.VMEM((_B,), _I32),
            pltpu.VMEM((_B, dp), _F32),
            pltpu.VMEM((rpt, dp), _F32),
            pltpu.VMEM_SHARED((rng, dp), _F32),
        ],
    )
    def ks(msg_hbm, coll_hbm, goff_hbm, out_hbm,
           offs_v, cl_v, rows_v, zbuf, acc_sh):
        cid = lax.axis_index("c")
        sid = lax.axis_index("s")
        wid = sid * _NC + cid
        zero16 = jnp.zeros((_L,), _F32)

        pltpu.sync_copy(goff_hbm, offs_v)
        offs = offs_v[...]

        @pl.loop(0, rpt)
        def _(i):
            for kk in range(dp // _L):
                zbuf[i, pl.ds(kk * _L, _L)] = zero16

        for r in range(n_rng):
            pltpu.sync_copy(zbuf, acc_sh.at[pl.ds(sid * rpt, rpt)])
            plsc.subcore_barrier()
            b0 = offs[r] // _B
            b1 = offs[r + 1] // _B

            @pl.loop(b0 + wid, b1, step=_NW)
            def _(b):
                off = b * _B
                pltpu.sync_copy(coll_hbm.at[pl.ds(off, _B)], cl_v)
                pltpu.sync_copy(msg_hbm.at[pl.ds(off, _B)], rows_v)
                pltpu.sync_copy(rows_v, acc_sh.at[cl_v], add=True)

            plsc.subcore_barrier()
            pltpu.sync_copy(
                acc_sh.at[pl.ds(sid * rpt, rpt)],
                out_hbm.at[cid, pl.ds(r * rng + sid * rpt, rpt)])
            plsc.subcore_barrier()

    return ks


_BLKE = 2048


def _tc_edge_coeff(ec, er):
    ep = ec.shape[0]

    def body(ec_ref, er_ref, msg_ref):
        xc = ec_ref[:, :128]
        tt = ec_ref[:, 128:256]
        gg = ec_ref[:, 256:384]
        xj = er_ref[:, :128]
        xm = er_ref[:, 128:256]
        df = xj - xc
        dd = df * df
        ss = jnp.sum(dd, axis=-1, keepdims=True)
        tn = jnp.sum(tt * dd, axis=-1, keepdims=True)
        u = xc * xj
        wi = jnp.sum(gg * u, axis=-1, keepdims=True)
        ni = jnp.sum(gg * xc * xc, axis=-1, keepdims=True)
        nj = jnp.sum(gg * xj * xj, axis=-1, keepdims=True)
        tau = tn / jnp.maximum(ss, 1e-16)
        den = jnp.sqrt(jnp.maximum(ni * nj, 1e-30)) + 1e-8
        alpha = jax.nn.sigmoid(wi / den)
        msg_ref[...] = tau * alpha * xm

    return pl.pallas_call(
        body,
        grid=(ep // _BLKE,),
        in_specs=[
            pl.BlockSpec((_BLKE, 384), lambda i: (i, 0)),
            pl.BlockSpec((_BLKE, 256), lambda i: (i, 0)),
        ],
        out_specs=pl.BlockSpec((_BLKE, 128), lambda i: (i, 0)),
        out_shape=jax.ShapeDtypeStruct((ep, 128), _F32),
    )(ec, er)


def _ln(h, g, b):
    mu = jnp.mean(h, axis=-1, keepdims=True)
    var = jnp.mean((h - mu) * (h - mu), axis=-1, keepdims=True)
    return (h - mu) / jnp.sqrt(var + 1e-5) * g + b


_BLK = 1024


def _tc_encoder(xp, p):
    n_pad = xp.shape[0]

    def body(x_ref, w_ref, b_ref, g_ref, bt_ref, o_ref):
        h = jnp.dot(x_ref[...], w_ref[...],
                    preferred_element_type=_F32) + b_ref[...]
        o_ref[...] = jnp.maximum(_ln(h, g_ref[...], bt_ref[...]), 0.0)

    d_in = xp.shape[1]
    hid = p['enc_W'].shape[1]
    return pl.pallas_call(
        body,
        grid=(n_pad // _BLK,),
        in_specs=[
            pl.BlockSpec((_BLK, d_in), lambda i: (i, 0)),
            pl.BlockSpec((d_in, hid), lambda i: (0, 0)),
            pl.BlockSpec((hid,), lambda i: (0,)),
            pl.BlockSpec((hid,), lambda i: (0,)),
            pl.BlockSpec((hid,), lambda i: (0,)),
        ],
        out_specs=pl.BlockSpec((_BLK, hid), lambda i: (i, 0)),
        out_shape=jax.ShapeDtypeStruct((n_pad, hid), _F32),
    )(xp, p['enc_W'], p['enc_b'], p['enc_g'], p['enc_bt'])


def _tc_dense(h, ns2, deg2, l, dout):
    n_pad = h.shape[0]
    rw = 256

    def body(h_ref, ns_ref, deg_ref, mw1_ref, mb1_ref, mg_ref, mbt_ref,
             mw2_ref, mb2_ref, msgw_ref, selfw_ref, selfb_ref,
             c_ref, r_ref, so_ref):
        hh = h_ref[...]
        s = ns_ref[0] + ns_ref[1] + hh
        cnt = deg_ref[...] + 1.0
        x_nb = s / jnp.maximum(cnt, 1.0)[:, None]
        comb = jnp.concatenate([hh, x_nb], axis=-1)
        hm = jnp.dot(comb, mw1_ref[...],
                     preferred_element_type=_F32) + mb1_ref[...]
        hm = jnp.maximum(_ln(hm, mg_ref[...], mbt_ref[...]), 0.0)
        raw = jnp.dot(hm, mw2_ref[...],
                      preferred_element_type=_F32) + mb2_ref[...]
        z2 = 2.0 * raw
        sp = jnp.maximum(z2, 0.0) + jnp.log1p(jnp.exp(-jnp.abs(z2)))
        g = jnp.clip(sp * 0.5, 0.001, 10.0)
        t = (1.0 - g * g) / (1.0 + g * g)
        xm = jnp.dot(hh, msgw_ref[...], preferred_element_type=_F32)
        so = jnp.dot(hh, selfw_ref[...],
                     preferred_element_type=_F32) + selfb_ref[...]
        c_ref[...] = jnp.concatenate([hh, t, g], axis=-1)
        if dout == 128:
            r_ref[...] = jnp.concatenate([hh, xm], axis=-1)
        else:
            pad = jnp.zeros((_BLK, 128 - dout), _F32)
            r_ref[...] = jnp.concatenate([hh, xm, pad], axis=-1)
        so_ref[...] = so

    return pl.pallas_call(
        body,
        grid=(n_pad // _BLK,),
        in_specs=[
            pl.BlockSpec((_BLK, 128), lambda i: (i, 0)),
            pl.BlockSpec((2, _BLK, 128), lambda i: (0, i, 0)),
            pl.BlockSpec((_BLK,), lambda i: (i,)),
            pl.BlockSpec((256, 64), lambda i: (0, 0)),
            pl.BlockSpec((64,), lambda i: (0,)),
            pl.BlockSpec((64,), lambda i: (0,)),
            pl.BlockSpec((64,), lambda i: (0,)),
            pl.BlockSpec((64, 128), lambda i: (0, 0)),
            pl.BlockSpec((128,), lambda i: (0,)),
            pl.BlockSpec((128, dout), lambda i: (0, 0)),
            pl.BlockSpec((128, dout), lambda i: (0, 0)),
            pl.BlockSpec((dout,), lambda i: (0,)),
        ],
        out_specs=[
            pl.BlockSpec((_BLK, 384), lambda i: (i, 0)),
            pl.BlockSpec((_BLK, rw), lambda i: (i, 0)),
            pl.BlockSpec((_BLK, dout), lambda i: (i, 0)),
        ],
        out_shape=[
            jax.ShapeDtypeStruct((n_pad, 384), _F32),
            jax.ShapeDtypeStruct((n_pad, rw), _F32),
            jax.ShapeDtypeStruct((n_pad, dout), _F32),
        ],
    )(h, ns2, deg2, l['mW1'], l['mb1'], l['mg'], l['mbt'],
      l['mW2'], l['mb2'], l['msgW'], l['selfW'], l['selfb'])


def _tc_epilogue(agg2, so, l, last):
    n_pad = so.shape[0]
    dout = so.shape[1]

    def body(agg_ref, so_ref, g_ref, bt_ref, o_ref):
        agg = agg_ref[0][:, :dout] + agg_ref[1][:, :dout]
        o = _ln(agg + so_ref[...], g_ref[...], bt_ref[...])
        if last:
            m = jnp.max(o, axis=-1, keepdims=True)
            lse = jnp.log(jnp.sum(jnp.exp(o - m), axis=-1, keepdims=True)) + m
            o_ref[...] = o - lse
        else:
            o_ref[...] = jnp.maximum(o, 0.0)

    return pl.pallas_call(
        body,
        grid=(n_pad // _BLK,),
        in_specs=[
            pl.BlockSpec((2, _BLK, 128), lambda i: (0, i, 0)),
            pl.BlockSpec((_BLK, dout), lambda i: (i, 0)),
            pl.BlockSpec((dout,), lambda i: (0,)),
            pl.BlockSpec((dout,), lambda i: (0,)),
        ],
        out_specs=pl.BlockSpec((_BLK, dout), lambda i: (i, 0)),
        out_shape=jax.ShapeDtypeStruct((n_pad, dout), _F32),
    )(agg2, so, l['ng'], l['nbt'])


def kernel(x, params, edge_index):
    n, d_in = x.shape
    e = edge_index.shape[1]

    n_rng = 8
    rng = _NS * 80
    n_acc = n_rng * rng
    row = edge_index[0]
    col = edge_index[1]

    grp = col // rng
    pos = jnp.zeros((e,), _I32)
    goffs = []
    base = jnp.int32(0)
    pad_counts = []
    for r in range(n_rng):
        m = grp == r
        ranks = jnp.cumsum(m.astype(_I32))
        size = ranks[-1]
        goffs.append(base)
        pos = jnp.where(m, base + ranks - 1, pos)
        padded = ((size + _B - 1) // _B) * _B
        pad_counts.append(padded - size)
        base = base + padded
    goffs.append(base)
    bpt = (e + n_rng * _B + _NW * _B - 1) // (_NW * _B)
    ep = bpt * _NW * _B
    rowp = jnp.full((ep,), n, _I32).at[pos].set(row)
    colp = jnp.zeros((ep,), _I32).at[pos].set(col - grp * rng)
    colg = jnp.full((ep,), n, _I32).at[pos].set(col)
    goff = jnp.zeros((16,), _I32).at[:n_rng + 1].set(jnp.stack(goffs))
    padvec = jnp.zeros((n_acc,), _F32).at[
        jnp.arange(n_rng) * rng].set(jnp.stack(pad_counts).astype(_F32))

    xp = jnp.pad(x, ((0, n_acc - n), (0, 0)))

    deg_out = _make_deg_kernel(n_acc, n_rng, rng)(colp, goff)
    deg = deg_out[0, :, 0] + deg_out[1, :, 0] - padvec
    h = _tc_encoder(xp, params)

    nsum = _make_nsum_kernel(n_acc, 128, n_rng, rng)
    gatherk = _make_gather_kernel(ep, bpt)
    scatterk = _make_scatter_kernel(n_acc, n_rng, rng)
    n_layers = len(params['layers'])
    for i, l in enumerate(params['layers']):
        dout = l['msgW'].shape[1]
        ns2 = nsum(h, rowp, colp, goff)
        c_tbl, r_tbl, so = _tc_dense(h, ns2, deg, l, dout)
        ec, er = gatherk(c_tbl, r_tbl, rowp, colg)
        msg = _tc_edge_coeff(ec, er)
        agg2 = scatterk(msg, colp, goff)
        h = _tc_epilogue(agg2, so, l, last=(i == n_layers - 1))

    return h[:n]


# speedup vs baseline: 1.9565x; 1.1049x over previous
import dataclasses
import functools

import jax
import jax.numpy as jnp
from jax import lax
from jax.experimental import pallas as pl
from jax.experimental.pallas import tpu as pltpu
from jax.experimental.pallas import tpu_sc as plsc

_NC = 2
_NS = 16
_NW = _NC * _NS
_L = 16
_B = 128

_F32 = jnp.float32
_I32 = jnp.int32


def _sc_compiler_params():
    cp = pltpu.CompilerParams()
    if "needs_layout_passes" in pltpu.CompilerParams.__dataclass_fields__:
        cp = dataclasses.replace(cp, needs_layout_passes=False)
    return cp


def _vector_mesh():
    return plsc.VectorSubcoreMesh(core_axis_name="c", subcore_axis_name="s",
                                  num_cores=_NC, num_subcores=_NS)


def _make_deg_kernel(n_acc, n_rng, rng):
    rpt = rng // _NS

    @functools.partial(
        pl.kernel,
        out_type=jax.ShapeDtypeStruct((_NC, n_acc, 128), _F32),
        mesh=_vector_mesh(),
        scratch_types=[
            pltpu.VMEM((16,), _I32),
            pltpu.VMEM((_B,), _I32),
            pltpu.VMEM((_B, 128), _F32),
            pltpu.VMEM((rpt, 128), _F32),
            pltpu.VMEM_SHARED((rng, 128), _F32),
        ],
    )
    def kd(coli_hbm, goff_hbm, out_hbm, offs_v, ci_v, ones_v, zbuf, acc_sh):
        cid = lax.axis_index("c")
        sid = lax.axis_index("s")
        wid = sid * _NC + cid
        zero16 = jnp.zeros((_L,), _F32)
        e0 = jnp.where(lax.iota(_I32, _L) == 0, 1.0, 0.0).astype(_F32)

        pltpu.sync_copy(goff_hbm, offs_v)
        offs = offs_v[...]

        @pl.loop(0, rpt)
        def _(i):
            for kk in range(128 // _L):
                zbuf[i, pl.ds(kk * _L, _L)] = zero16

        @pl.loop(0, _B)
        def _(i):
            ones_v[i, pl.ds(0, _L)] = e0
            for kk in range(1, 128 // _L):
                ones_v[i, pl.ds(kk * _L, _L)] = zero16

        for r in range(n_rng):
            pltpu.sync_copy(zbuf, acc_sh.at[pl.ds(sid * rpt, rpt)])
            plsc.subcore_barrier()
            b0 = offs[r] // _B
            b1 = offs[r + 1] // _B

            @pl.loop(b0 + wid, b1, step=_NW)
            def _(b):
                pltpu.sync_copy(coli_hbm.at[pl.ds(b * _B, _B)], ci_v)
                pltpu.sync_copy(ones_v, acc_sh.at[ci_v], add=True)

            plsc.subcore_barrier()
            pltpu.sync_copy(
                acc_sh.at[pl.ds(sid * rpt, rpt)],
                out_hbm.at[cid, pl.ds(r * rng + sid * rpt, rpt)])
            plsc.subcore_barrier()

    return kd


def _make_nsum_kernel(n_acc, width, n_rng, rng):
    rpt = rng // _NS

    @functools.partial(
        pl.kernel,
        out_type=jax.ShapeDtypeStruct((_NC, n_acc, width), _F32),
        mesh=_vector_mesh(),
        scratch_types=[
            pltpu.VMEM((16,), _I32),
            pltpu.VMEM((_B,), _I32),
            pltpu.VMEM((_B,), _I32),
            pltpu.VMEM((_B, width), _F32),
            pltpu.VMEM((rpt, width), _F32),
            pltpu.VMEM_SHARED((rng, width), _F32),
        ],
    )
    def ka(tbl_hbm, rowi_hbm, coli_hbm, goff_hbm, out_hbm,
           offs_v, ri_v, ci_v, rows_v, zbuf, acc_sh):
        cid = lax.axis_index("c")
        sid = lax.axis_index("s")
        wid = sid * _NC + cid
        zero16 = jnp.zeros((_L,), _F32)

        pltpu.sync_copy(goff_hbm, offs_v)
        offs = offs_v[...]

        @pl.loop(0, rpt)
        def _(i):
            for kk in range(width // _L):
                zbuf[i, pl.ds(kk * _L, _L)] = zero16

        for r in range(n_rng):
            pltpu.sync_copy(zbuf, acc_sh.at[pl.ds(sid * rpt, rpt)])
            plsc.subcore_barrier()
            b0 = offs[r] // _B
            b1 = offs[r + 1] // _B

            @pl.loop(b0 + wid, b1, step=_NW)
            def _(b):
                off = b * _B
                pltpu.sync_copy(rowi_hbm.at[pl.ds(off, _B)], ri_v)
                pltpu.sync_copy(coli_hbm.at[pl.ds(off, _B)], ci_v)
                pltpu.sync_copy(tbl_hbm.at[ri_v], rows_v)
                pltpu.sync_copy(rows_v, acc_sh.at[ci_v], add=True)

            plsc.subcore_barrier()
            pltpu.sync_copy(
                acc_sh.at[pl.ds(sid * rpt, rpt)],
                out_hbm.at[cid, pl.ds(r * rng + sid * rpt, rpt)])
            plsc.subcore_barrier()

    return ka


def _make_gather_kernel(ep):
    bg = 64
    bpt = ep // (_NW * bg)

    @functools.partial(
        pl.kernel,
        out_type=[
            jax.ShapeDtypeStruct((ep, 384), _F32),
            jax.ShapeDtypeStruct((ep, 256), _F32),
        ],
        mesh=_vector_mesh(),
        scratch_types=[
            pltpu.VMEM((bg,), _I32),
            pltpu.VMEM((bg,), _I32),
            pltpu.VMEM((bg,), _I32),
            pltpu.VMEM((bg,), _I32),
            pltpu.VMEM((bg, 384), _F32),
            pltpu.VMEM((bg, 384), _F32),
            pltpu.VMEM((bg, 256), _F32),
            pltpu.VMEM((bg, 256), _F32),
        ] + [pltpu.SemaphoreType.DMA] * 12,
    )
    def kg(c_hbm, r_hbm, rowi_hbm, colg_hbm, ec_hbm, er_hbm,
           ri0, ri1, cg0, cg1, cr0, cr1, rr0, rr1,
           s0, s1, s2, s3, s4, s5, s6, s7, s8, s9, s10, s11):
        cid = lax.axis_index("c")
        sid = lax.axis_index("s")
        wid = sid * _NC + cid
        ebase = wid * bpt * bg

        @pl.loop(0, bpt, step=2)
        def _(b):
            off0 = ebase + b * bg
            off1 = off0 + bg
            di0r = pltpu.async_copy(rowi_hbm.at[pl.ds(off0, bg)], ri0, s0)
            di0c = pltpu.async_copy(colg_hbm.at[pl.ds(off0, bg)], cg0, s1)
            di1r = pltpu.async_copy(rowi_hbm.at[pl.ds(off1, bg)], ri1, s2)
            di1c = pltpu.async_copy(colg_hbm.at[pl.ds(off1, bg)], cg1, s3)
            di0c.wait()
            g0c = pltpu.async_copy(c_hbm.at[cg0], cr0, s4)
            di0r.wait()
            g0r = pltpu.async_copy(r_hbm.at[ri0], rr0, s5)
            di1c.wait()
            g1c = pltpu.async_copy(c_hbm.at[cg1], cr1, s6)
            di1r.wait()
            g1r = pltpu.async_copy(r_hbm.at[ri1], rr1, s7)
            g0c.wait()
            o0c = pltpu.async_copy(cr0, ec_hbm.at[pl.ds(off0, bg)], s8)
            g0r.wait()
            o0r = pltpu.async_copy(rr0, er_hbm.at[pl.ds(off0, bg)], s9)
            g1c.wait()
            o1c = pltpu.async_copy(cr1, ec_hbm.at[pl.ds(off1, bg)], s10)
            g1r.wait()
            o1r = pltpu.async_copy(rr1, er_hbm.at[pl.ds(off1, bg)], s11)
            o0c.wait()
            o0r.wait()
            o1c.wait()
            o1r.wait()

    return kg


def _make_scatter_kernel(n_acc, n_rng, rng):
    rpt = rng // _NS
    dp = 128

    @functools.partial(
        pl.kernel,
        out_type=jax.ShapeDtypeStruct((_NC, n_acc, dp), _F32),
        mesh=_vector_mesh(),
        scratch_types=[
            pltpu.VMEM((16,), _I32),
            pltpu.VMEM((_B,), _I32),
            pltpu.VMEM((_B, dp), _F32),
            pltpu.VMEM((rpt, dp), _F32),
            pltpu.VMEM_SHARED((rng, dp), _F32),
        ],
    )
    def ks(msg_hbm, coll_hbm, goff_hbm, out_hbm,
           offs_v, cl_v, rows_v, zbuf, acc_sh):
        cid = lax.axis_index("c")
        sid = lax.axis_index("s")
        wid = sid * _NC + cid
        zero16 = jnp.zeros((_L,), _F32)

        pltpu.sync_copy(goff_hbm, offs_v)
        offs = offs_v[...]

        @pl.loop(0, rpt)
        def _(i):
            for kk in range(dp // _L):
                zbuf[i, pl.ds(kk * _L, _L)] = zero16

        for r in range(n_rng):
            pltpu.sync_copy(zbuf, acc_sh.at[pl.ds(sid * rpt, rpt)])
            plsc.subcore_barrier()
            b0 = offs[r] // _B
            b1 = offs[r + 1] // _B

            @pl.loop(b0 + wid, b1, step=_NW)
            def _(b):
                off = b * _B
                pltpu.sync_copy(coll_hbm.at[pl.ds(off, _B)], cl_v)
                pltpu.sync_copy(msg_hbm.at[pl.ds(off, _B)], rows_v)
                pltpu.sync_copy(rows_v, acc_sh.at[cl_v], add=True)

            plsc.subcore_barrier()
            pltpu.sync_copy(
                acc_sh.at[pl.ds(sid * rpt, rpt)],
                out_hbm.at[cid, pl.ds(r * rng + sid * rpt, rpt)])
            plsc.subcore_barrier()

    return ks


_BLKE = 2048


def _tc_edge_coeff(ec, er):
    ep = ec.shape[0]

    def body(ec_ref, er_ref, msg_ref):
        xc = ec_ref[:, :128]
        tt = ec_ref[:, 128:256]
        gg = ec_ref[:, 256:384]
        xj = er_ref[:, :128]
        xm = er_ref[:, 128:256]
        df = xj - xc
        dd = df * df
        ss = jnp.sum(dd, axis=-1, keepdims=True)
        tn = jnp.sum(tt * dd, axis=-1, keepdims=True)
        u = xc * xj
        wi = jnp.sum(gg * u, axis=-1, keepdims=True)
        ni = jnp.sum(gg * xc * xc, axis=-1, keepdims=True)
        nj = jnp.sum(gg * xj * xj, axis=-1, keepdims=True)
        tau = tn / jnp.maximum(ss, 1e-16)
        den = jnp.sqrt(jnp.maximum(ni * nj, 1e-30)) + 1e-8
        alpha = jax.nn.sigmoid(wi / den)
        msg_ref[...] = tau * alpha * xm

    return pl.pallas_call(
        body,
        grid=(ep // _BLKE,),
        in_specs=[
            pl.BlockSpec((_BLKE, 384), lambda i: (i, 0)),
            pl.BlockSpec((_BLKE, 256), lambda i: (i, 0)),
        ],
        out_specs=pl.BlockSpec((_BLKE, 128), lambda i: (i, 0)),
        out_shape=jax.ShapeDtypeStruct((ep, 128), _F32),
    )(ec, er)


def _ln(h, g, b):
    mu = jnp.mean(h, axis=-1, keepdims=True)
    var = jnp.mean((h - mu) * (h - mu), axis=-1, keepdims=True)
    return (h - mu) / jnp.sqrt(var + 1e-5) * g + b


_BLK = 1024


def _tc_encoder(xp, p):
    n_pad = xp.shape[0]

    def body(x_ref, w_ref, b_ref, g_ref, bt_ref, o_ref):
        h = jnp.dot(x_ref[...], w_ref[...],
                    preferred_element_type=_F32) + b_ref[...]
        o_ref[...] = jnp.maximum(_ln(h, g_ref[...], bt_ref[...]), 0.0)

    d_in = xp.shape[1]
    hid = p['enc_W'].shape[1]
    return pl.pallas_call(
        body,
        grid=(n_pad // _BLK,),
        in_specs=[
            pl.BlockSpec((_BLK, d_in), lambda i: (i, 0)),
            pl.BlockSpec((d_in, hid), lambda i: (0, 0)),
            pl.BlockSpec((hid,), lambda i: (0,)),
            pl.BlockSpec((hid,), lambda i: (0,)),
            pl.BlockSpec((hid,), lambda i: (0,)),
        ],
        out_specs=pl.BlockSpec((_BLK, hid), lambda i: (i, 0)),
        out_shape=jax.ShapeDtypeStruct((n_pad, hid), _F32),
    )(xp, p['enc_W'], p['enc_b'], p['enc_g'], p['enc_bt'])


def _tc_dense(h, ns2, deg2, l, dout):
    n_pad = h.shape[0]
    rw = 256

    def body(h_ref, ns_ref, deg_ref, mw1_ref, mb1_ref, mg_ref, mbt_ref,
             mw2_ref, mb2_ref, msgw_ref, selfw_ref, selfb_ref,
             c_ref, r_ref, so_ref):
        hh = h_ref[...]
        s = ns_ref[0] + ns_ref[1] + hh
        cnt = deg_ref[...] + 1.0
        x_nb = s / jnp.maximum(cnt, 1.0)[:, None]
        comb = jnp.concatenate([hh, x_nb], axis=-1)
        hm = jnp.dot(comb, mw1_ref[...],
                     preferred_element_type=_F32) + mb1_ref[...]
        hm = jnp.maximum(_ln(hm, mg_ref[...], mbt_ref[...]), 0.0)
        raw = jnp.dot(hm, mw2_ref[...],
                      preferred_element_type=_F32) + mb2_ref[...]
        z2 = 2.0 * raw
        sp = jnp.maximum(z2, 0.0) + jnp.log1p(jnp.exp(-jnp.abs(z2)))
        g = jnp.clip(sp * 0.5, 0.001, 10.0)
        t = (1.0 - g * g) / (1.0 + g * g)
        xm = jnp.dot(hh, msgw_ref[...], preferred_element_type=_F32)
        so = jnp.dot(hh, selfw_ref[...],
                     preferred_element_type=_F32) + selfb_ref[...]
        c_ref[...] = jnp.concatenate([hh, t, g], axis=-1)
        if dout == 128:
            r_ref[...] = jnp.concatenate([hh, xm], axis=-1)
        else:
            pad = jnp.zeros((_BLK, 128 - dout), _F32)
            r_ref[...] = jnp.concatenate([hh, xm, pad], axis=-1)
        so_ref[...] = so

    return pl.pallas_call(
        body,
        grid=(n_pad // _BLK,),
        in_specs=[
            pl.BlockSpec((_BLK, 128), lambda i: (i, 0)),
            pl.BlockSpec((2, _BLK, 128), lambda i: (0, i, 0)),
            pl.BlockSpec((_BLK,), lambda i: (i,)),
            pl.BlockSpec((256, 64), lambda i: (0, 0)),
            pl.BlockSpec((64,), lambda i: (0,)),
            pl.BlockSpec((64,), lambda i: (0,)),
            pl.BlockSpec((64,), lambda i: (0,)),
            pl.BlockSpec((64, 128), lambda i: (0, 0)),
            pl.BlockSpec((128,), lambda i: (0,)),
            pl.BlockSpec((128, dout), lambda i: (0, 0)),
            pl.BlockSpec((128, dout), lambda i: (0, 0)),
            pl.BlockSpec((dout,), lambda i: (0,)),
        ],
        out_specs=[
            pl.BlockSpec((_BLK, 384), lambda i: (i, 0)),
            pl.BlockSpec((_BLK, rw), lambda i: (i, 0)),
            pl.BlockSpec((_BLK, dout), lambda i: (i, 0)),
        ],
        out_shape=[
            jax.ShapeDtypeStruct((n_pad, 384), _F32),
            jax.ShapeDtypeStruct((n_pad, rw), _F32),
            jax.ShapeDtypeStruct((n_pad, dout), _F32),
        ],
    )(h, ns2, deg2, l['mW1'], l['mb1'], l['mg'], l['mbt'],
      l['mW2'], l['mb2'], l['msgW'], l['selfW'], l['selfb'])


def _tc_epilogue(agg2, so, l, last):
    n_pad = so.shape[0]
    dout = so.shape[1]

    def body(agg_ref, so_ref, g_ref, bt_ref, o_ref):
        agg = agg_ref[0][:, :dout] + agg_ref[1][:, :dout]
        o = _ln(agg + so_ref[...], g_ref[...], bt_ref[...])
        if last:
            m = jnp.max(o, axis=-1, keepdims=True)
            lse = jnp.log(jnp.sum(jnp.exp(o - m), axis=-1, keepdims=True)) + m
            o_ref[...] = o - lse
        else:
            o_ref[...] = jnp.maximum(o, 0.0)

    return pl.pallas_call(
        body,
        grid=(n_pad // _BLK,),
        in_specs=[
            pl.BlockSpec((2, _BLK, 128), lambda i: (0, i, 0)),
            pl.BlockSpec((_BLK, dout), lambda i: (i, 0)),
            pl.BlockSpec((dout,), lambda i: (0,)),
            pl.BlockSpec((dout,), lambda i: (0,)),
        ],
        out_specs=pl.BlockSpec((_BLK, dout), lambda i: (i, 0)),
        out_shape=jax.ShapeDtypeStruct((n_pad, dout), _F32),
    )(agg2, so, l['ng'], l['nbt'])


def kernel(x, params, edge_index):
    n, d_in = x.shape
    e = edge_index.shape[1]

    n_rng = 8
    rng = _NS * 80
    n_acc = n_rng * rng
    row = edge_index[0]
    col = edge_index[1]

    grp = col // rng
    pos = jnp.zeros((e,), _I32)
    goffs = []
    base = jnp.int32(0)
    pad_counts = []
    for r in range(n_rng):
        m = grp == r
        ranks = jnp.cumsum(m.astype(_I32))
        size = ranks[-1]
        goffs.append(base)
        pos = jnp.where(m, base + ranks - 1, pos)
        padded = ((size + _B - 1) // _B) * _B
        pad_counts.append(padded - size)
        base = base + padded
    goffs.append(base)
    bpt = (e + n_rng * _B + _NW * _B - 1) // (_NW * _B)
    ep = bpt * _NW * _B
    rowp = jnp.full((ep,), n, _I32).at[pos].set(row)
    colp = jnp.zeros((ep,), _I32).at[pos].set(col - grp * rng)
    colg = jnp.full((ep,), n, _I32).at[pos].set(col)
    goff = jnp.zeros((16,), _I32).at[:n_rng + 1].set(jnp.stack(goffs))
    padvec = jnp.zeros((n_acc,), _F32).at[
        jnp.arange(n_rng) * rng].set(jnp.stack(pad_counts).astype(_F32))

    xp = jnp.pad(x, ((0, n_acc - n), (0, 0)))

    deg_out = _make_deg_kernel(n_acc, n_rng, rng)(colp, goff)
    deg = deg_out[0, :, 0] + deg_out[1, :, 0] - padvec
    h = _tc_encoder(xp, params)

    nsum = _make_nsum_kernel(n_acc, 128, n_rng, rng)
    gatherk = _make_gather_kernel(ep)
    scatterk = _make_scatter_kernel(n_acc, n_rng, rng)
    n_layers = len(params['layers'])
    for i, l in enumerate(params['layers']):
        dout = l['msgW'].shape[1]
        ns2 = nsum(h, rowp, colp, goff)
        c_tbl, r_tbl, so = _tc_dense(h, ns2, deg, l, dout)
        ec, er = gatherk(c_tbl, r_tbl, rowp, colg)
        msg = _tc_edge_coeff(ec, er)
        agg2 = scatterk(msg, colp, goff)
        h = _tc_epilogue(agg2, so, l, last=(i == n_layers - 1))

    return h[:n]


# speedup vs baseline: 2.0417x; 1.0435x over previous
import dataclasses
import functools

import jax
import jax.numpy as jnp
from jax import lax
from jax.experimental import pallas as pl
from jax.experimental.pallas import tpu as pltpu
from jax.experimental.pallas import tpu_sc as plsc

_NC = 2
_NS = 16
_NW = _NC * _NS
_L = 16
_B = 128

_F32 = jnp.float32
_I32 = jnp.int32


def _sc_compiler_params():
    cp = pltpu.CompilerParams()
    if "needs_layout_passes" in pltpu.CompilerParams.__dataclass_fields__:
        cp = dataclasses.replace(cp, needs_layout_passes=False)
    return cp


def _vector_mesh():
    return plsc.VectorSubcoreMesh(core_axis_name="c", subcore_axis_name="s",
                                  num_cores=_NC, num_subcores=_NS)


def _make_deg_kernel(n_acc, n_rng, rng):
    rpt = rng // _NS

    @functools.partial(
        pl.kernel,
        out_type=jax.ShapeDtypeStruct((_NC, n_acc, 128), _F32),
        mesh=_vector_mesh(),
        scratch_types=[
            pltpu.VMEM((16,), _I32),
            pltpu.VMEM((_B,), _I32),
            pltpu.VMEM((_B, 128), _F32),
            pltpu.VMEM((rpt, 128), _F32),
            pltpu.VMEM_SHARED((rng, 128), _F32),
        ],
    )
    def kd(coli_hbm, goff_hbm, out_hbm, offs_v, ci_v, ones_v, zbuf, acc_sh):
        cid = lax.axis_index("c")
        sid = lax.axis_index("s")
        wid = sid * _NC + cid
        zero16 = jnp.zeros((_L,), _F32)
        e0 = jnp.where(lax.iota(_I32, _L) == 0, 1.0, 0.0).astype(_F32)

        pltpu.sync_copy(goff_hbm, offs_v)
        offs = offs_v[...]

        @pl.loop(0, rpt)
        def _(i):
            for kk in range(128 // _L):
                zbuf[i, pl.ds(kk * _L, _L)] = zero16

        @pl.loop(0, _B)
        def _(i):
            ones_v[i, pl.ds(0, _L)] = e0
            for kk in range(1, 128 // _L):
                ones_v[i, pl.ds(kk * _L, _L)] = zero16

        for r in range(n_rng):
            pltpu.sync_copy(zbuf, acc_sh.at[pl.ds(sid * rpt, rpt)])
            plsc.subcore_barrier()
            b0 = offs[r] // _B
            b1 = offs[r + 1] // _B

            @pl.loop(b0 + wid, b1, step=_NW)
            def _(b):
                pltpu.sync_copy(coli_hbm.at[pl.ds(b * _B, _B)], ci_v)
                pltpu.sync_copy(ones_v, acc_sh.at[ci_v], add=True)

            plsc.subcore_barrier()
            pltpu.sync_copy(
                acc_sh.at[pl.ds(sid * rpt, rpt)],
                out_hbm.at[cid, pl.ds(r * rng + sid * rpt, rpt)])
            plsc.subcore_barrier()

    return kd


def _make_nsum_kernel(n_acc, width, n_rng, rng):
    rpt = rng // _NS

    @functools.partial(
        pl.kernel,
        out_type=jax.ShapeDtypeStruct((_NC, n_acc, width), _F32),
        mesh=_vector_mesh(),
        scratch_types=[
            pltpu.VMEM((16,), _I32),
            pltpu.VMEM((_B,), _I32),
            pltpu.VMEM((_B,), _I32),
            pltpu.VMEM((_B,), _I32),
            pltpu.VMEM((_B,), _I32),
            pltpu.VMEM((_B, width), _F32),
            pltpu.VMEM((_B, width), _F32),
            pltpu.VMEM((rpt, width), _F32),
            pltpu.VMEM_SHARED((rng, width), _F32),
        ] + [pltpu.SemaphoreType.DMA] * 8,
    )
    def ka(tbl_hbm, rowi_hbm, coli_hbm, goff_hbm, out_hbm,
           offs_v, riA, ciA, riB, ciB, rowsA, rowsB, zbuf, acc_sh,
           s0, s1, s2, s3, s4, s5, s6, s7):
        cid = lax.axis_index("c")
        sid = lax.axis_index("s")
        wid = sid * _NC + cid
        zero16 = jnp.zeros((_L,), _F32)

        pltpu.sync_copy(goff_hbm, offs_v)
        offs = offs_v[...]

        @pl.loop(0, rpt)
        def _(i):
            for kk in range(width // _L):
                zbuf[i, pl.ds(kk * _L, _L)] = zero16

        for r in range(n_rng):
            pltpu.sync_copy(zbuf, acc_sh.at[pl.ds(sid * rpt, rpt)])
            plsc.subcore_barrier()
            b0 = offs[r] // _B
            b1 = offs[r + 1] // _B

            @pl.loop(b0 + wid, b1, step=2 * _NW)
            def _(b):
                offA = b * _B
                offB = (b + _NW) * _B
                hb = b + _NW < b1
                dAr = pltpu.async_copy(rowi_hbm.at[pl.ds(offA, _B)], riA, s0)
                dAc = pltpu.async_copy(coli_hbm.at[pl.ds(offA, _B)], ciA, s1)

                @pl.when(hb)
                def _():
                    pltpu.async_copy(rowi_hbm.at[pl.ds(offB, _B)], riB, s2)
                    pltpu.async_copy(coli_hbm.at[pl.ds(offB, _B)], ciB, s3)

                dAr.wait()
                gA = pltpu.async_copy(tbl_hbm.at[riA], rowsA, s4)

                @pl.when(hb)
                def _():
                    pltpu.make_async_copy(
                        rowi_hbm.at[pl.ds(offB, _B)], riB, s2).wait()
                    pltpu.async_copy(tbl_hbm.at[riB], rowsB, s5)

                gA.wait()
                dAc.wait()
                sA = pltpu.async_copy(rowsA, acc_sh.at[ciA], add=True, sem=s6)

                @pl.when(hb)
                def _():
                    pltpu.make_async_copy(tbl_hbm.at[riB], rowsB, s5).wait()
                    pltpu.make_async_copy(
                        coli_hbm.at[pl.ds(offB, _B)], ciB, s3).wait()
                    pltpu.async_copy(rowsB, acc_sh.at[ciB], add=True, sem=s7)

                sA.wait()

                @pl.when(hb)
                def _():
                    pltpu.make_async_copy(rowsB, acc_sh.at[ciB], s7).wait()

            plsc.subcore_barrier()
            pltpu.sync_copy(
                acc_sh.at[pl.ds(sid * rpt, rpt)],
                out_hbm.at[cid, pl.ds(r * rng + sid * rpt, rpt)])
            plsc.subcore_barrier()

    return ka


def _make_gather_kernel(ep):
    bg = 64
    bpt = ep // (_NW * bg)

    @functools.partial(
        pl.kernel,
        out_type=[
            jax.ShapeDtypeStruct((ep, 384), _F32),
            jax.ShapeDtypeStruct((ep, 256), _F32),
        ],
        mesh=_vector_mesh(),
        scratch_types=[
            pltpu.VMEM((bg,), _I32),
            pltpu.VMEM((bg,), _I32),
            pltpu.VMEM((bg,), _I32),
            pltpu.VMEM((bg,), _I32),
            pltpu.VMEM((bg, 384), _F32),
            pltpu.VMEM((bg, 384), _F32),
            pltpu.VMEM((bg, 256), _F32),
            pltpu.VMEM((bg, 256), _F32),
        ] + [pltpu.SemaphoreType.DMA] * 12,
    )
    def kg(c_hbm, r_hbm, rowi_hbm, colg_hbm, ec_hbm, er_hbm,
           ri0, ri1, cg0, cg1, cr0, cr1, rr0, rr1,
           s0, s1, s2, s3, s4, s5, s6, s7, s8, s9, s10, s11):
        cid = lax.axis_index("c")
        sid = lax.axis_index("s")
        wid = sid * _NC + cid
        ebase = wid * bpt * bg

        @pl.loop(0, bpt, step=2)
        def _(b):
            off0 = ebase + b * bg
            off1 = off0 + bg
            di0r = pltpu.async_copy(rowi_hbm.at[pl.ds(off0, bg)], ri0, s0)
            di0c = pltpu.async_copy(colg_hbm.at[pl.ds(off0, bg)], cg0, s1)
            di1r = pltpu.async_copy(rowi_hbm.at[pl.ds(off1, bg)], ri1, s2)
            di1c = pltpu.async_copy(colg_hbm.at[pl.ds(off1, bg)], cg1, s3)
            di0c.wait()
            g0c = pltpu.async_copy(c_hbm.at[cg0], cr0, s4)
            di0r.wait()
            g0r = pltpu.async_copy(r_hbm.at[ri0], rr0, s5)
            di1c.wait()
            g1c = pltpu.async_copy(c_hbm.at[cg1], cr1, s6)
            di1r.wait()
            g1r = pltpu.async_copy(r_hbm.at[ri1], rr1, s7)
            g0c.wait()
            o0c = pltpu.async_copy(cr0, ec_hbm.at[pl.ds(off0, bg)], s8)
            g0r.wait()
            o0r = pltpu.async_copy(rr0, er_hbm.at[pl.ds(off0, bg)], s9)
            g1c.wait()
            o1c = pltpu.async_copy(cr1, ec_hbm.at[pl.ds(off1, bg)], s10)
            g1r.wait()
            o1r = pltpu.async_copy(rr1, er_hbm.at[pl.ds(off1, bg)], s11)
            o0c.wait()
            o0r.wait()
            o1c.wait()
            o1r.wait()

    return kg


def _make_scatter_kernel(n_acc, n_rng, rng):
    rpt = rng // _NS
    dp = 128

    @functools.partial(
        pl.kernel,
        out_type=jax.ShapeDtypeStruct((_NC, n_acc, dp), _F32),
        mesh=_vector_mesh(),
        scratch_types=[
            pltpu.VMEM((16,), _I32),
            pltpu.VMEM((_B,), _I32),
            pltpu.VMEM((_B,), _I32),
            pltpu.VMEM((_B, dp), _F32),
            pltpu.VMEM((_B, dp), _F32),
            pltpu.VMEM((rpt, dp), _F32),
            pltpu.VMEM_SHARED((rng, dp), _F32),
        ] + [pltpu.SemaphoreType.DMA] * 6,
    )
    def ks(msg_hbm, coll_hbm, goff_hbm, out_hbm,
           offs_v, clA, clB, rowsA, rowsB, zbuf, acc_sh,
           s0, s1, s2, s3, s4, s5):
        cid = lax.axis_index("c")
        sid = lax.axis_index("s")
        wid = sid * _NC + cid
        zero16 = jnp.zeros((_L,), _F32)

        pltpu.sync_copy(goff_hbm, offs_v)
        offs = offs_v[...]

        @pl.loop(0, rpt)
        def _(i):
            for kk in range(dp // _L):
                zbuf[i, pl.ds(kk * _L, _L)] = zero16

        for r in range(n_rng):
            pltpu.sync_copy(zbuf, acc_sh.at[pl.ds(sid * rpt, rpt)])
            plsc.subcore_barrier()
            b0 = offs[r] // _B
            b1 = offs[r + 1] // _B

            @pl.loop(b0 + wid, b1, step=2 * _NW)
            def _(b):
                offA = b * _B
                offB = (b + _NW) * _B
                hb = b + _NW < b1
                dAc = pltpu.async_copy(coll_hbm.at[pl.ds(offA, _B)], clA, s0)
                dAm = pltpu.async_copy(msg_hbm.at[pl.ds(offA, _B)], rowsA, s1)

                @pl.when(hb)
                def _():
                    pltpu.async_copy(coll_hbm.at[pl.ds(offB, _B)], clB, s2)
                    pltpu.async_copy(msg_hbm.at[pl.ds(offB, _B)], rowsB, s3)

                dAc.wait()
                dAm.wait()
                sA = pltpu.async_copy(rowsA, acc_sh.at[clA], s4, add=True)

                @pl.when(hb)
                def _():
                    pltpu.make_async_copy(
                        coll_hbm.at[pl.ds(offB, _B)], clB, s2).wait()
                    pltpu.make_async_copy(
                        msg_hbm.at[pl.ds(offB, _B)], rowsB, s3).wait()
                    pltpu.async_copy(rowsB, acc_sh.at[clB], s5, add=True)

                sA.wait()

                @pl.when(hb)
                def _():
                    pltpu.make_async_copy(rowsB, acc_sh.at[clB], s5).wait()

            plsc.subcore_barrier()
            pltpu.sync_copy(
                acc_sh.at[pl.ds(sid * rpt, rpt)],
                out_hbm.at[cid, pl.ds(r * rng + sid * rpt, rpt)])
            plsc.subcore_barrier()

    return ks


_BLKE = 2048


def _tc_edge_coeff(ec, er):
    ep = ec.shape[0]

    def body(ec_ref, er_ref, msg_ref):
        xc = ec_ref[:, :128]
        tt = ec_ref[:, 128:256]
        gg = ec_ref[:, 256:384]
        xj = er_ref[:, :128]
        xm = er_ref[:, 128:256]
        df = xj - xc
        dd = df * df
        ss = jnp.sum(dd, axis=-1, keepdims=True)
        tn = jnp.sum(tt * dd, axis=-1, keepdims=True)
        u = xc * xj
        wi = jnp.sum(gg * u, axis=-1, keepdims=True)
        ni = jnp.sum(gg * xc * xc, axis=-1, keepdims=True)
        nj = jnp.sum(gg * xj * xj, axis=-1, keepdims=True)
        tau = tn / jnp.maximum(ss, 1e-16)
        den = jnp.sqrt(jnp.maximum(ni * nj, 1e-30)) + 1e-8
        alpha = jax.nn.sigmoid(wi / den)
        msg_ref[...] = tau * alpha * xm

    return pl.pallas_call(
        body,
        grid=(ep // _BLKE,),
        in_specs=[
            pl.BlockSpec((_BLKE, 384), lambda i: (i, 0)),
            pl.BlockSpec((_BLKE, 256), lambda i: (i, 0)),
        ],
        out_specs=pl.BlockSpec((_BLKE, 128), lambda i: (i, 0)),
        out_shape=jax.ShapeDtypeStruct((ep, 128), _F32),
    )(ec, er)


def _ln(h, g, b):
    mu = jnp.mean(h, axis=-1, keepdims=True)
    var = jnp.mean((h - mu) * (h - mu), axis=-1, keepdims=True)
    return (h - mu) / jnp.sqrt(var + 1e-5) * g + b


_BLK = 1024


def _tc_encoder(xp, p):
    n_pad = xp.shape[0]

    def body(x_ref, w_ref, b_ref, g_ref, bt_ref, o_ref):
        h = jnp.dot(x_ref[...], w_ref[...],
                    preferred_element_type=_F32) + b_ref[...]
        o_ref[...] = jnp.maximum(_ln(h, g_ref[...], bt_ref[...]), 0.0)

    d_in = xp.shape[1]
    hid = p['enc_W'].shape[1]
    return pl.pallas_call(
        body,
        grid=(n_pad // _BLK,),
        in_specs=[
            pl.BlockSpec((_BLK, d_in), lambda i: (i, 0)),
            pl.BlockSpec((d_in, hid), lambda i: (0, 0)),
            pl.BlockSpec((hid,), lambda i: (0,)),
            pl.BlockSpec((hid,), lambda i: (0,)),
            pl.BlockSpec((hid,), lambda i: (0,)),
        ],
        out_specs=pl.BlockSpec((_BLK, hid), lambda i: (i, 0)),
        out_shape=jax.ShapeDtypeStruct((n_pad, hid), _F32),
    )(xp, p['enc_W'], p['enc_b'], p['enc_g'], p['enc_bt'])


def _tc_dense(h, ns2, deg2, l, dout):
    n_pad = h.shape[0]
    rw = 256

    def body(h_ref, ns_ref, deg_ref, mw1_ref, mb1_ref, mg_ref, mbt_ref,
             mw2_ref, mb2_ref, msgw_ref, selfw_ref, selfb_ref,
             c_ref, r_ref, so_ref):
        hh = h_ref[...]
        s = ns_ref[0] + ns_ref[1] + hh
        cnt = deg_ref[...] + 1.0
        x_nb = s / jnp.maximum(cnt, 1.0)[:, None]
        comb = jnp.concatenate([hh, x_nb], axis=-1)
        hm = jnp.dot(comb, mw1_ref[...],
                     preferred_element_type=_F32) + mb1_ref[...]
        hm = jnp.maximum(_ln(hm, mg_ref[...], mbt_ref[...]), 0.0)
        raw = jnp.dot(hm, mw2_ref[...],
                      preferred_element_type=_F32) + mb2_ref[...]
        z2 = 2.0 * raw
        sp = jnp.maximum(z2, 0.0) + jnp.log1p(jnp.exp(-jnp.abs(z2)))
        g = jnp.clip(sp * 0.5, 0.001, 10.0)
        t = (1.0 - g * g) / (1.0 + g * g)
        xm = jnp.dot(hh, msgw_ref[...], preferred_element_type=_F32)
        so = jnp.dot(hh, selfw_ref[...],
                     preferred_element_type=_F32) + selfb_ref[...]
        c_ref[...] = jnp.concatenate([hh, t, g], axis=-1)
        if dout == 128:
            r_ref[...] = jnp.concatenate([hh, xm], axis=-1)
        else:
            pad = jnp.zeros((_BLK, 128 - dout), _F32)
            r_ref[...] = jnp.concatenate([hh, xm, pad], axis=-1)
        so_ref[...] = so

    return pl.pallas_call(
        body,
        grid=(n_pad // _BLK,),
        in_specs=[
            pl.BlockSpec((_BLK, 128), lambda i: (i, 0)),
            pl.BlockSpec((2, _BLK, 128), lambda i: (0, i, 0)),
            pl.BlockSpec((_BLK,), lambda i: (i,)),
            pl.BlockSpec((256, 64), lambda i: (0, 0)),
            pl.BlockSpec((64,), lambda i: (0,)),
            pl.BlockSpec((64,), lambda i: (0,)),
            pl.BlockSpec((64,), lambda i: (0,)),
            pl.BlockSpec((64, 128), lambda i: (0, 0)),
            pl.BlockSpec((128,), lambda i: (0,)),
            pl.BlockSpec((128, dout), lambda i: (0, 0)),
            pl.BlockSpec((128, dout), lambda i: (0, 0)),
            pl.BlockSpec((dout,), lambda i: (0,)),
        ],
        out_specs=[
            pl.BlockSpec((_BLK, 384), lambda i: (i, 0)),
            pl.BlockSpec((_BLK, rw), lambda i: (i, 0)),
            pl.BlockSpec((_BLK, dout), lambda i: (i, 0)),
        ],
        out_shape=[
            jax.ShapeDtypeStruct((n_pad, 384), _F32),
            jax.ShapeDtypeStruct((n_pad, rw), _F32),
            jax.ShapeDtypeStruct((n_pad, dout), _F32),
        ],
    )(h, ns2, deg2, l['mW1'], l['mb1'], l['mg'], l['mbt'],
      l['mW2'], l['mb2'], l['msgW'], l['selfW'], l['selfb'])


def _tc_epilogue(agg2, so, l, last):
    n_pad = so.shape[0]
    dout = so.shape[1]

    def body(agg_ref, so_ref, g_ref, bt_ref, o_ref):
        agg = agg_ref[0][:, :dout] + agg_ref[1][:, :dout]
        o = _ln(agg + so_ref[...], g_ref[...], bt_ref[...])
        if last:
            m = jnp.max(o, axis=-1, keepdims=True)
            lse = jnp.log(jnp.sum(jnp.exp(o - m), axis=-1, keepdims=True)) + m
            o_ref[...] = o - lse
        else:
            o_ref[...] = jnp.maximum(o, 0.0)

    return pl.pallas_call(
        body,
        grid=(n_pad // _BLK,),
        in_specs=[
            pl.BlockSpec((2, _BLK, 128), lambda i: (0, i, 0)),
            pl.BlockSpec((_BLK, dout), lambda i: (i, 0)),
            pl.BlockSpec((dout,), lambda i: (0,)),
            pl.BlockSpec((dout,), lambda i: (0,)),
        ],
        out_specs=pl.BlockSpec((_BLK, dout), lambda i: (i, 0)),
        out_shape=jax.ShapeDtypeStruct((n_pad, dout), _F32),
    )(agg2, so, l['ng'], l['nbt'])


def kernel(x, params, edge_index):
    n, d_in = x.shape
    e = edge_index.shape[1]

    n_rng = 8
    rng = _NS * 80
    n_acc = n_rng * rng
    row = edge_index[0]
    col = edge_index[1]

    grp = col // rng
    pos = jnp.zeros((e,), _I32)
    goffs = []
    base = jnp.int32(0)
    pad_counts = []
    for r in range(n_rng):
        m = grp == r
        ranks = jnp.cumsum(m.astype(_I32))
        size = ranks[-1]
        goffs.append(base)
        pos = jnp.where(m, base + ranks - 1, pos)
        padded = ((size + _B - 1) // _B) * _B
        pad_counts.append(padded - size)
        base = base + padded
    goffs.append(base)
    bpt = (e + n_rng * _B + _NW * _B - 1) // (_NW * _B)
    ep = bpt * _NW * _B
    rowp = jnp.full((ep,), n, _I32).at[pos].set(row)
    colp = jnp.zeros((ep,), _I32).at[pos].set(col - grp * rng)
    colg = jnp.full((ep,), n, _I32).at[pos].set(col)
    goff = jnp.zeros((16,), _I32).at[:n_rng + 1].set(jnp.stack(goffs))
    padvec = jnp.zeros((n_acc,), _F32).at[
        jnp.arange(n_rng) * rng].set(jnp.stack(pad_counts).astype(_F32))

    xp = jnp.pad(x, ((0, n_acc - n), (0, 0)))

    deg_out = _make_deg_kernel(n_acc, n_rng, rng)(colp, goff)
    deg = deg_out[0, :, 0] + deg_out[1, :, 0] - padvec
    h = _tc_encoder(xp, params)

    nsum = _make_nsum_kernel(n_acc, 128, n_rng, rng)
    gatherk = _make_gather_kernel(ep)
    scatterk = _make_scatter_kernel(n_acc, n_rng, rng)
    n_layers = len(params['layers'])
    for i, l in enumerate(params['layers']):
        dout = l['msgW'].shape[1]
        ns2 = nsum(h, rowp, colp, goff)
        c_tbl, r_tbl, so = _tc_dense(h, ns2, deg, l, dout)
        ec, er = gatherk(c_tbl, r_tbl, rowp, colg)
        msg = _tc_edge_coeff(ec, er)
        agg2 = scatterk(msg, colp, goff)
        h = _tc_epilogue(agg2, so, l, last=(i == n_layers - 1))

    return h[:n]


# speedup vs baseline: 2.1621x; 1.0590x over previous
import dataclasses
import functools

import jax
import jax.numpy as jnp
from jax import lax
from jax.experimental import pallas as pl
from jax.experimental.pallas import tpu as pltpu
from jax.experimental.pallas import tpu_sc as plsc

_NC = 2
_NS = 16
_NW = _NC * _NS
_L = 16
_B = 128

_F32 = jnp.float32
_I32 = jnp.int32


def _sc_compiler_params():
    cp = pltpu.CompilerParams()
    if "needs_layout_passes" in pltpu.CompilerParams.__dataclass_fields__:
        cp = dataclasses.replace(cp, needs_layout_passes=False)
    return cp


def _vector_mesh():
    return plsc.VectorSubcoreMesh(core_axis_name="c", subcore_axis_name="s",
                                  num_cores=_NC, num_subcores=_NS)


def _make_deg_kernel(n_acc, n_rng, rng):
    rpt = rng // _NS

    @functools.partial(
        pl.kernel,
        out_type=jax.ShapeDtypeStruct((_NC, n_acc, 128), _F32),
        mesh=_vector_mesh(),
        scratch_types=[
            pltpu.VMEM((16,), _I32),
            pltpu.VMEM((_B,), _I32),
            pltpu.VMEM((_B, 128), _F32),
            pltpu.VMEM((rpt, 128), _F32),
            pltpu.VMEM_SHARED((rng, 128), _F32),
        ],
    )
    def kd(coli_hbm, goff_hbm, out_hbm, offs_v, ci_v, ones_v, zbuf, acc_sh):
        cid = lax.axis_index("c")
        sid = lax.axis_index("s")
        wid = sid * _NC + cid
        zero16 = jnp.zeros((_L,), _F32)
        e0 = jnp.where(lax.iota(_I32, _L) == 0, 1.0, 0.0).astype(_F32)

        pltpu.sync_copy(goff_hbm, offs_v)
        offs = offs_v[...]

        @pl.loop(0, rpt)
        def _(i):
            for kk in range(128 // _L):
                zbuf[i, pl.ds(kk * _L, _L)] = zero16

        @pl.loop(0, _B)
        def _(i):
            ones_v[i, pl.ds(0, _L)] = e0
            for kk in range(1, 128 // _L):
                ones_v[i, pl.ds(kk * _L, _L)] = zero16

        for r in range(n_rng):
            pltpu.sync_copy(zbuf, acc_sh.at[pl.ds(sid * rpt, rpt)])
            plsc.subcore_barrier()
            b0 = offs[r] // _B
            b1 = offs[r + 1] // _B

            @pl.loop(b0 + wid, b1, step=_NW)
            def _(b):
                pltpu.sync_copy(coli_hbm.at[pl.ds(b * _B, _B)], ci_v)
                pltpu.sync_copy(ones_v, acc_sh.at[ci_v], add=True)

            plsc.subcore_barrier()
            pltpu.sync_copy(
                acc_sh.at[pl.ds(sid * rpt, rpt)],
                out_hbm.at[cid, pl.ds(r * rng + sid * rpt, rpt)])
            plsc.subcore_barrier()

    return kd


def _make_nsum_kernel(n_acc, width, n_rng, rng):
    rpt = rng // _NS

    @functools.partial(
        pl.kernel,
        out_type=jax.ShapeDtypeStruct((_NC, n_acc, width), _F32),
        mesh=_vector_mesh(),
        scratch_types=[
            pltpu.VMEM((16,), _I32),
            pltpu.VMEM((_B,), _I32),
            pltpu.VMEM((_B,), _I32),
            pltpu.VMEM((_B,), _I32),
            pltpu.VMEM((_B,), _I32),
            pltpu.VMEM((_B, width), _F32),
            pltpu.VMEM((_B, width), _F32),
            pltpu.VMEM((rpt, width), _F32),
            pltpu.VMEM_SHARED((rng, width), _F32),
        ] + [pltpu.SemaphoreType.DMA] * 8,
    )
    def ka(tbl_hbm, rowi_hbm, coli_hbm, goff_hbm, out_hbm,
           offs_v, riA, ciA, riB, ciB, rowsA, rowsB, zbuf, acc_sh,
           s0, s1, s2, s3, s4, s5, s6, s7):
        cid = lax.axis_index("c")
        sid = lax.axis_index("s")
        wid = sid * _NC + cid
        zero16 = jnp.zeros((_L,), _F32)

        pltpu.sync_copy(goff_hbm, offs_v)
        offs = offs_v[...]

        @pl.loop(0, rpt)
        def _(i):
            for kk in range(width // _L):
                zbuf[i, pl.ds(kk * _L, _L)] = zero16

        for r in range(n_rng):
            pltpu.sync_copy(zbuf, acc_sh.at[pl.ds(sid * rpt, rpt)])
            plsc.subcore_barrier()
            b0 = offs[r] // _B
            b1 = offs[r + 1] // _B

            @pl.loop(b0 + wid, b1, step=2 * _NW)
            def _(b):
                offA = b * _B
                offB = (b + _NW) * _B
                hb = b + _NW < b1
                dAr = pltpu.async_copy(rowi_hbm.at[pl.ds(offA, _B)], riA, s0)
                dAc = pltpu.async_copy(coli_hbm.at[pl.ds(offA, _B)], ciA, s1)

                @pl.when(hb)
                def _():
                    pltpu.async_copy(rowi_hbm.at[pl.ds(offB, _B)], riB, s2)
                    pltpu.async_copy(coli_hbm.at[pl.ds(offB, _B)], ciB, s3)

                dAr.wait()
                gA = pltpu.async_copy(tbl_hbm.at[riA], rowsA, s4)

                @pl.when(hb)
                def _():
                    pltpu.make_async_copy(
                        rowi_hbm.at[pl.ds(offB, _B)], riB, s2).wait()
                    pltpu.async_copy(tbl_hbm.at[riB], rowsB, s5)

                gA.wait()
                dAc.wait()
                sA = pltpu.async_copy(rowsA, acc_sh.at[ciA], add=True, sem=s6)

                @pl.when(hb)
                def _():
                    pltpu.make_async_copy(tbl_hbm.at[riB], rowsB, s5).wait()
                    pltpu.make_async_copy(
                        coli_hbm.at[pl.ds(offB, _B)], ciB, s3).wait()
                    pltpu.async_copy(rowsB, acc_sh.at[ciB], add=True, sem=s7)

                sA.wait()

                @pl.when(hb)
                def _():
                    pltpu.make_async_copy(rowsB, acc_sh.at[ciB], s7).wait()

            plsc.subcore_barrier()
            pltpu.sync_copy(
                acc_sh.at[pl.ds(sid * rpt, rpt)],
                out_hbm.at[cid, pl.ds(r * rng + sid * rpt, rpt)])
            plsc.subcore_barrier()

    return ka


def _pack_pairs(a, nrows):
    ia = lax.bitcast_convert_type(a, _I32)
    lo = lax.shift_right_logical(ia[:, :64] + jnp.int32(0x8000), 16)
    hi = lax.shift_right_logical(ia[:, 64:] + jnp.int32(0x8000), 16)
    word = jnp.bitwise_or(lax.shift_left(hi, 16), lo)
    return lax.bitcast_convert_type(word, _F32)


def _unpack_pairs(w, nrows):
    iw = lax.bitcast_convert_type(w, _I32)
    lo = lax.bitcast_convert_type(lax.shift_left(iw, 16), _F32)
    hi = lax.bitcast_convert_type(
        jnp.bitwise_and(iw, jnp.int32(-65536)), _F32)
    return jnp.concatenate([lo, hi], axis=-1)


def _make_gather_kernel(ep):
    bg = 64
    bpt = ep // (_NW * bg)

    @functools.partial(
        pl.kernel,
        out_type=[
            jax.ShapeDtypeStruct((ep, 256), _F32),
            jax.ShapeDtypeStruct((ep, 128), _F32),
        ],
        mesh=_vector_mesh(),
        scratch_types=[
            pltpu.VMEM((bg,), _I32),
            pltpu.VMEM((bg,), _I32),
            pltpu.VMEM((bg,), _I32),
            pltpu.VMEM((bg,), _I32),
            pltpu.VMEM((bg, 256), _F32),
            pltpu.VMEM((bg, 256), _F32),
            pltpu.VMEM((bg, 128), _F32),
            pltpu.VMEM((bg, 128), _F32),
        ] + [pltpu.SemaphoreType.DMA] * 12,
    )
    def kg(c_hbm, r_hbm, rowi_hbm, colg_hbm, ec_hbm, er_hbm,
           ri0, ri1, cg0, cg1, cr0, cr1, rr0, rr1,
           s0, s1, s2, s3, s4, s5, s6, s7, s8, s9, s10, s11):
        cid = lax.axis_index("c")
        sid = lax.axis_index("s")
        wid = sid * _NC + cid
        ebase = wid * bpt * bg

        @pl.loop(0, bpt, step=2)
        def _(b):
            off0 = ebase + b * bg
            off1 = off0 + bg
            di0r = pltpu.async_copy(rowi_hbm.at[pl.ds(off0, bg)], ri0, s0)
            di0c = pltpu.async_copy(colg_hbm.at[pl.ds(off0, bg)], cg0, s1)
            di1r = pltpu.async_copy(rowi_hbm.at[pl.ds(off1, bg)], ri1, s2)
            di1c = pltpu.async_copy(colg_hbm.at[pl.ds(off1, bg)], cg1, s3)
            di0c.wait()
            g0c = pltpu.async_copy(c_hbm.at[cg0], cr0, s4)
            di0r.wait()
            g0r = pltpu.async_copy(r_hbm.at[ri0], rr0, s5)
            di1c.wait()
            g1c = pltpu.async_copy(c_hbm.at[cg1], cr1, s6)
            di1r.wait()
            g1r = pltpu.async_copy(r_hbm.at[ri1], rr1, s7)
            g0c.wait()
            o0c = pltpu.async_copy(cr0, ec_hbm.at[pl.ds(off0, bg)], s8)
            g0r.wait()
            o0r = pltpu.async_copy(rr0, er_hbm.at[pl.ds(off0, bg)], s9)
            g1c.wait()
            o1c = pltpu.async_copy(cr1, ec_hbm.at[pl.ds(off1, bg)], s10)
            g1r.wait()
            o1r = pltpu.async_copy(rr1, er_hbm.at[pl.ds(off1, bg)], s11)
            o0c.wait()
            o0r.wait()
            o1c.wait()
            o1r.wait()

    return kg


def _make_scatter_kernel(n_acc, n_rng, rng):
    rpt = rng // _NS
    dp = 128

    @functools.partial(
        pl.kernel,
        out_type=jax.ShapeDtypeStruct((_NC, n_acc, dp), _F32),
        mesh=_vector_mesh(),
        scratch_types=[
            pltpu.VMEM((16,), _I32),
            pltpu.VMEM((_B,), _I32),
            pltpu.VMEM((_B,), _I32),
            pltpu.VMEM((_B, dp), _F32),
            pltpu.VMEM((_B, dp), _F32),
            pltpu.VMEM((rpt, dp), _F32),
            pltpu.VMEM_SHARED((rng, dp), _F32),
        ] + [pltpu.SemaphoreType.DMA] * 6,
    )
    def ks(msg_hbm, coll_hbm, goff_hbm, out_hbm,
           offs_v, clA, clB, rowsA, rowsB, zbuf, acc_sh,
           s0, s1, s2, s3, s4, s5):
        cid = lax.axis_index("c")
        sid = lax.axis_index("s")
        wid = sid * _NC + cid
        zero16 = jnp.zeros((_L,), _F32)

        pltpu.sync_copy(goff_hbm, offs_v)
        offs = offs_v[...]

        @pl.loop(0, rpt)
        def _(i):
            for kk in range(dp // _L):
                zbuf[i, pl.ds(kk * _L, _L)] = zero16

        for r in range(n_rng):
            pltpu.sync_copy(zbuf, acc_sh.at[pl.ds(sid * rpt, rpt)])
            plsc.subcore_barrier()
            b0 = offs[r] // _B
            b1 = offs[r + 1] // _B

            @pl.loop(b0 + wid, b1, step=2 * _NW)
            def _(b):
                offA = b * _B
                offB = (b + _NW) * _B
                hb = b + _NW < b1
                dAc = pltpu.async_copy(coll_hbm.at[pl.ds(offA, _B)], clA, s0)
                dAm = pltpu.async_copy(msg_hbm.at[pl.ds(offA, _B)], rowsA, s1)

                @pl.when(hb)
                def _():
                    pltpu.async_copy(coll_hbm.at[pl.ds(offB, _B)], clB, s2)
                    pltpu.async_copy(msg_hbm.at[pl.ds(offB, _B)], rowsB, s3)

                dAc.wait()
                dAm.wait()
                sA = pltpu.async_copy(rowsA, acc_sh.at[clA], s4, add=True)

                @pl.when(hb)
                def _():
                    pltpu.make_async_copy(
                        coll_hbm.at[pl.ds(offB, _B)], clB, s2).wait()
                    pltpu.make_async_copy(
                        msg_hbm.at[pl.ds(offB, _B)], rowsB, s3).wait()
                    pltpu.async_copy(rowsB, acc_sh.at[clB], s5, add=True)

                sA.wait()

                @pl.when(hb)
                def _():
                    pltpu.make_async_copy(rowsB, acc_sh.at[clB], s5).wait()

            plsc.subcore_barrier()
            pltpu.sync_copy(
                acc_sh.at[pl.ds(sid * rpt, rpt)],
                out_hbm.at[cid, pl.ds(r * rng + sid * rpt, rpt)])
            plsc.subcore_barrier()

    return ks


_BLKE = 2048


def _tc_edge_coeff(ec, er):
    ep = ec.shape[0]

    def body(ec_ref, er_ref, msg_ref):
        xc = _unpack_pairs(ec_ref[:, :64], _BLKE)
        tt = _unpack_pairs(ec_ref[:, 64:128], _BLKE)
        gg = _unpack_pairs(ec_ref[:, 128:192], _BLKE)
        xj = _unpack_pairs(er_ref[:, :64], _BLKE)
        xm = _unpack_pairs(er_ref[:, 64:128], _BLKE)
        df = xj - xc
        dd = df * df
        ss = jnp.sum(dd, axis=-1, keepdims=True)
        tn = jnp.sum(tt * dd, axis=-1, keepdims=True)
        u = xc * xj
        wi = jnp.sum(gg * u, axis=-1, keepdims=True)
        ni = jnp.sum(gg * xc * xc, axis=-1, keepdims=True)
        nj = jnp.sum(gg * xj * xj, axis=-1, keepdims=True)
        tau = tn / jnp.maximum(ss, 1e-16)
        den = jnp.sqrt(jnp.maximum(ni * nj, 1e-30)) + 1e-8
        alpha = jax.nn.sigmoid(wi / den)
        msg_ref[...] = tau * alpha * xm

    return pl.pallas_call(
        body,
        grid=(ep // _BLKE,),
        in_specs=[
            pl.BlockSpec((_BLKE, 256), lambda i: (i, 0)),
            pl.BlockSpec((_BLKE, 128), lambda i: (i, 0)),
        ],
        out_specs=pl.BlockSpec((_BLKE, 128), lambda i: (i, 0)),
        out_shape=jax.ShapeDtypeStruct((ep, 128), _F32),
    )(ec, er)


def _ln(h, g, b):
    mu = jnp.mean(h, axis=-1, keepdims=True)
    var = jnp.mean((h - mu) * (h - mu), axis=-1, keepdims=True)
    return (h - mu) / jnp.sqrt(var + 1e-5) * g + b


_BLK = 1024


def _tc_encoder(xp, p):
    n_pad = xp.shape[0]

    def body(x_ref, w_ref, b_ref, g_ref, bt_ref, o_ref):
        h = jnp.dot(x_ref[...], w_ref[...],
                    preferred_element_type=_F32) + b_ref[...]
        o_ref[...] = jnp.maximum(_ln(h, g_ref[...], bt_ref[...]), 0.0)

    d_in = xp.shape[1]
    hid = p['enc_W'].shape[1]
    return pl.pallas_call(
        body,
        grid=(n_pad // _BLK,),
        in_specs=[
            pl.BlockSpec((_BLK, d_in), lambda i: (i, 0)),
            pl.BlockSpec((d_in, hid), lambda i: (0, 0)),
            pl.BlockSpec((hid,), lambda i: (0,)),
            pl.BlockSpec((hid,), lambda i: (0,)),
            pl.BlockSpec((hid,), lambda i: (0,)),
        ],
        out_specs=pl.BlockSpec((_BLK, hid), lambda i: (i, 0)),
        out_shape=jax.ShapeDtypeStruct((n_pad, hid), _F32),
    )(xp, p['enc_W'], p['enc_b'], p['enc_g'], p['enc_bt'])


def _tc_dense(h, ns2, deg2, l, dout):
    n_pad = h.shape[0]
    rw = 256

    def body(h_ref, ns_ref, deg_ref, mw1_ref, mb1_ref, mg_ref, mbt_ref,
             mw2_ref, mb2_ref, msgw_ref, selfw_ref, selfb_ref,
             c_ref, r_ref, so_ref):
        hh = h_ref[...]
        s = ns_ref[0] + ns_ref[1] + hh
        cnt = deg_ref[...] + 1.0
        x_nb = s / jnp.maximum(cnt, 1.0)[:, None]
        comb = jnp.concatenate([hh, x_nb], axis=-1)
        hm = jnp.dot(comb, mw1_ref[...],
                     preferred_element_type=_F32) + mb1_ref[...]
        hm = jnp.maximum(_ln(hm, mg_ref[...], mbt_ref[...]), 0.0)
        raw = jnp.dot(hm, mw2_ref[...],
                      preferred_element_type=_F32) + mb2_ref[...]
        z2 = 2.0 * raw
        sp = jnp.maximum(z2, 0.0) + jnp.log1p(jnp.exp(-jnp.abs(z2)))
        g = jnp.clip(sp * 0.5, 0.001, 10.0)
        t = (1.0 - g * g) / (1.0 + g * g)
        xm = jnp.dot(hh, msgw_ref[...], preferred_element_type=_F32)
        so = jnp.dot(hh, selfw_ref[...],
                     preferred_element_type=_F32) + selfb_ref[...]
        hp = _pack_pairs(hh, _BLK)
        tp = _pack_pairs(t, _BLK)
        gp = _pack_pairs(g, _BLK)
        zp = jnp.zeros((_BLK, 64), _F32)
        if dout == 128:
            xmp = _pack_pairs(xm, _BLK)
        else:
            xmp = _pack_pairs(
                jnp.concatenate([xm, jnp.zeros((_BLK, 128 - dout), _F32)],
                                axis=-1), _BLK)
        c_ref[...] = jnp.concatenate([hp, tp, gp, zp], axis=-1)
        r_ref[...] = jnp.concatenate([hp, xmp], axis=-1)
        so_ref[...] = so

    return pl.pallas_call(
        body,
        grid=(n_pad // _BLK,),
        in_specs=[
            pl.BlockSpec((_BLK, 128), lambda i: (i, 0)),
            pl.BlockSpec((2, _BLK, 128), lambda i: (0, i, 0)),
            pl.BlockSpec((_BLK,), lambda i: (i,)),
            pl.BlockSpec((256, 64), lambda i: (0, 0)),
            pl.BlockSpec((64,), lambda i: (0,)),
            pl.BlockSpec((64,), lambda i: (0,)),
            pl.BlockSpec((64,), lambda i: (0,)),
            pl.BlockSpec((64, 128), lambda i: (0, 0)),
            pl.BlockSpec((128,), lambda i: (0,)),
            pl.BlockSpec((128, dout), lambda i: (0, 0)),
            pl.BlockSpec((128, dout), lambda i: (0, 0)),
            pl.BlockSpec((dout,), lambda i: (0,)),
        ],
        out_specs=[
            pl.BlockSpec((_BLK, 256), lambda i: (i, 0)),
            pl.BlockSpec((_BLK, 128), lambda i: (i, 0)),
            pl.BlockSpec((_BLK, dout), lambda i: (i, 0)),
        ],
        out_shape=[
            jax.ShapeDtypeStruct((n_pad, 256), _F32),
            jax.ShapeDtypeStruct((n_pad, 128), _F32),
            jax.ShapeDtypeStruct((n_pad, dout), _F32),
        ],
    )(h, ns2, deg2, l['mW1'], l['mb1'], l['mg'], l['mbt'],
      l['mW2'], l['mb2'], l['msgW'], l['selfW'], l['selfb'])


def _tc_epilogue(agg2, so, l, last):
    n_pad = so.shape[0]
    dout = so.shape[1]

    def body(agg_ref, so_ref, g_ref, bt_ref, o_ref):
        agg = agg_ref[0][:, :dout] + agg_ref[1][:, :dout]
        o = _ln(agg + so_ref[...], g_ref[...], bt_ref[...])
        if last:
            m = jnp.max(o, axis=-1, keepdims=True)
            lse = jnp.log(jnp.sum(jnp.exp(o - m), axis=-1, keepdims=True)) + m
            o_ref[...] = o - lse
        else:
            o_ref[...] = jnp.maximum(o, 0.0)

    return pl.pallas_call(
        body,
        grid=(n_pad // _BLK,),
        in_specs=[
            pl.BlockSpec((2, _BLK, 128), lambda i: (0, i, 0)),
            pl.BlockSpec((_BLK, dout), lambda i: (i, 0)),
            pl.BlockSpec((dout,), lambda i: (0,)),
            pl.BlockSpec((dout,), lambda i: (0,)),
        ],
        out_specs=pl.BlockSpec((_BLK, dout), lambda i: (i, 0)),
        out_shape=jax.ShapeDtypeStruct((n_pad, dout), _F32),
    )(agg2, so, l['ng'], l['nbt'])


def kernel(x, params, edge_index):
    n, d_in = x.shape
    e = edge_index.shape[1]

    n_rng = 8
    rng = _NS * 80
    n_acc = n_rng * rng
    row = edge_index[0]
    col = edge_index[1]

    grp = col // rng
    pos = jnp.zeros((e,), _I32)
    goffs = []
    base = jnp.int32(0)
    pad_counts = []
    for r in range(n_rng):
        m = grp == r
        ranks = jnp.cumsum(m.astype(_I32))
        size = ranks[-1]
        goffs.append(base)
        pos = jnp.where(m, base + ranks - 1, pos)
        padded = ((size + _B - 1) // _B) * _B
        pad_counts.append(padded - size)
        base = base + padded
    goffs.append(base)
    bpt = (e + n_rng * _B + _NW * _B - 1) // (_NW * _B)
    ep = bpt * _NW * _B
    rowp = jnp.full((ep,), n, _I32).at[pos].set(row)
    colp = jnp.zeros((ep,), _I32).at[pos].set(col - grp * rng)
    colg = jnp.full((ep,), n, _I32).at[pos].set(col)
    goff = jnp.zeros((16,), _I32).at[:n_rng + 1].set(jnp.stack(goffs))
    padvec = jnp.zeros((n_acc,), _F32).at[
        jnp.arange(n_rng) * rng].set(jnp.stack(pad_counts).astype(_F32))

    xp = jnp.pad(x, ((0, n_acc - n), (0, 0)))

    deg_out = _make_deg_kernel(n_acc, n_rng, rng)(colp, goff)
    deg = deg_out[0, :, 0] + deg_out[1, :, 0] - padvec
    h = _tc_encoder(xp, params)

    nsum = _make_nsum_kernel(n_acc, 128, n_rng, rng)
    gatherk = _make_gather_kernel(ep)
    scatterk = _make_scatter_kernel(n_acc, n_rng, rng)
    n_layers = len(params['layers'])
    for i, l in enumerate(params['layers']):
        dout = l['msgW'].shape[1]
        ns2 = nsum(h, rowp, colp, goff)
        c_tbl, r_tbl, so = _tc_dense(h, ns2, deg, l, dout)
        ec, er = gatherk(c_tbl, r_tbl, rowp, colg)
        msg = _tc_edge_coeff(ec, er)
        agg2 = scatterk(msg, colp, goff)
        h = _tc_epilogue(agg2, so, l, last=(i == n_layers - 1))

    return h[:n]


# speedup vs baseline: 2.2007x; 1.0179x over previous
import dataclasses
import functools

import jax
import jax.numpy as jnp
from jax import lax
from jax.experimental import pallas as pl
from jax.experimental.pallas import tpu as pltpu
from jax.experimental.pallas import tpu_sc as plsc

_NC = 2
_NS = 16
_NW = _NC * _NS
_L = 16
_B = 128

_F32 = jnp.float32
_I32 = jnp.int32


def _sc_compiler_params():
    cp = pltpu.CompilerParams()
    if "needs_layout_passes" in pltpu.CompilerParams.__dataclass_fields__:
        cp = dataclasses.replace(cp, needs_layout_passes=False)
    return cp


def _vector_mesh():
    return plsc.VectorSubcoreMesh(core_axis_name="c", subcore_axis_name="s",
                                  num_cores=_NC, num_subcores=_NS)


def _make_deg_kernel(n_acc, n_rng, rng):
    rpt = rng // _NS

    @functools.partial(
        pl.kernel,
        out_type=jax.ShapeDtypeStruct((_NC, n_acc, 128), _F32),
        mesh=_vector_mesh(),
        scratch_types=[
            pltpu.VMEM((16,), _I32),
            pltpu.VMEM((_B,), _I32),
            pltpu.VMEM((_B, 128), _F32),
            pltpu.VMEM((rpt, 128), _F32),
            pltpu.VMEM_SHARED((rng, 128), _F32),
        ],
    )
    def kd(coli_hbm, goff_hbm, out_hbm, offs_v, ci_v, ones_v, zbuf, acc_sh):
        cid = lax.axis_index("c")
        sid = lax.axis_index("s")
        wid = sid * _NC + cid
        zero16 = jnp.zeros((_L,), _F32)
        e0 = jnp.where(lax.iota(_I32, _L) == 0, 1.0, 0.0).astype(_F32)

        pltpu.sync_copy(goff_hbm, offs_v)
        offs = offs_v[...]

        @pl.loop(0, rpt)
        def _(i):
            for kk in range(128 // _L):
                zbuf[i, pl.ds(kk * _L, _L)] = zero16

        @pl.loop(0, _B)
        def _(i):
            ones_v[i, pl.ds(0, _L)] = e0
            for kk in range(1, 128 // _L):
                ones_v[i, pl.ds(kk * _L, _L)] = zero16

        for r in range(n_rng):
            pltpu.sync_copy(zbuf, acc_sh.at[pl.ds(sid * rpt, rpt)])
            plsc.subcore_barrier()
            b0 = offs[r] // _B
            b1 = offs[r + 1] // _B

            @pl.loop(b0 + wid, b1, step=_NW)
            def _(b):
                pltpu.sync_copy(coli_hbm.at[pl.ds(b * _B, _B)], ci_v)
                pltpu.sync_copy(ones_v, acc_sh.at[ci_v], add=True)

            plsc.subcore_barrier()
            pltpu.sync_copy(
                acc_sh.at[pl.ds(sid * rpt, rpt)],
                out_hbm.at[cid, pl.ds(r * rng + sid * rpt, rpt)])
            plsc.subcore_barrier()

    return kd


def _make_nsum_kernel(n_acc, width, n_rng, rng):
    rpt = rng // _NS

    @functools.partial(
        pl.kernel,
        out_type=jax.ShapeDtypeStruct((_NC, n_acc, width), _F32),
        mesh=_vector_mesh(),
        scratch_types=[
            pltpu.VMEM((16,), _I32),
            pltpu.VMEM((_B,), _I32),
            pltpu.VMEM((_B,), _I32),
            pltpu.VMEM((_B,), _I32),
            pltpu.VMEM((_B,), _I32),
            pltpu.VMEM((_B, width), _F32),
            pltpu.VMEM((_B, width), _F32),
            pltpu.VMEM((rpt, width), _F32),
            pltpu.VMEM_SHARED((rng, width), _F32),
        ] + [pltpu.SemaphoreType.DMA] * 8,
    )
    def ka(tbl_hbm, rowi_hbm, coli_hbm, goff_hbm, out_hbm,
           offs_v, riA, ciA, riB, ciB, rowsA, rowsB, zbuf, acc_sh,
           s0, s1, s2, s3, s4, s5, s6, s7):
        cid = lax.axis_index("c")
        sid = lax.axis_index("s")
        wid = sid * _NC + cid
        zero16 = jnp.zeros((_L,), _F32)

        pltpu.sync_copy(goff_hbm, offs_v)
        offs = offs_v[...]

        @pl.loop(0, rpt)
        def _(i):
            for kk in range(width // _L):
                zbuf[i, pl.ds(kk * _L, _L)] = zero16

        for r in range(n_rng):
            pltpu.sync_copy(zbuf, acc_sh.at[pl.ds(sid * rpt, rpt)])
            plsc.subcore_barrier()
            b0 = offs[r] // _B
            b1 = offs[r + 1] // _B

            @pl.loop(b0 + wid, b1, step=2 * _NW)
            def _(b):
                offA = b * _B
                offB = (b + _NW) * _B
                hb = b + _NW < b1
                dAr = pltpu.async_copy(rowi_hbm.at[pl.ds(offA, _B)], riA, s0)
                dAc = pltpu.async_copy(coli_hbm.at[pl.ds(offA, _B)], ciA, s1)

                @pl.when(hb)
                def _():
                    pltpu.async_copy(rowi_hbm.at[pl.ds(offB, _B)], riB, s2)
                    pltpu.async_copy(coli_hbm.at[pl.ds(offB, _B)], ciB, s3)

                dAr.wait()
                gA = pltpu.async_copy(tbl_hbm.at[riA], rowsA, s4)

                @pl.when(hb)
                def _():
                    pltpu.make_async_copy(
                        rowi_hbm.at[pl.ds(offB, _B)], riB, s2).wait()
                    pltpu.async_copy(tbl_hbm.at[riB], rowsB, s5)

                gA.wait()
                dAc.wait()
                sA = pltpu.async_copy(rowsA, acc_sh.at[ciA], add=True, sem=s6)

                @pl.when(hb)
                def _():
                    pltpu.make_async_copy(tbl_hbm.at[riB], rowsB, s5).wait()
                    pltpu.make_async_copy(
                        coli_hbm.at[pl.ds(offB, _B)], ciB, s3).wait()
                    pltpu.async_copy(rowsB, acc_sh.at[ciB], add=True, sem=s7)

                sA.wait()

                @pl.when(hb)
                def _():
                    pltpu.make_async_copy(rowsB, acc_sh.at[ciB], s7).wait()

            plsc.subcore_barrier()
            pltpu.sync_copy(
                acc_sh.at[pl.ds(sid * rpt, rpt)],
                out_hbm.at[cid, pl.ds(r * rng + sid * rpt, rpt)])
            plsc.subcore_barrier()

    return ka


def _pack_pairs(a, nrows):
    ia = lax.bitcast_convert_type(a, _I32)
    lo = lax.shift_right_logical(ia[:, :64] + jnp.int32(0x8000), 16)
    hi = lax.shift_right_logical(ia[:, 64:] + jnp.int32(0x8000), 16)
    word = jnp.bitwise_or(lax.shift_left(hi, 16), lo)
    return lax.bitcast_convert_type(word, _F32)


def _unpack_pairs(w, nrows):
    iw = lax.bitcast_convert_type(w, _I32)
    lo = lax.bitcast_convert_type(lax.shift_left(iw, 16), _F32)
    hi = lax.bitcast_convert_type(
        jnp.bitwise_and(iw, jnp.int32(-65536)), _F32)
    return jnp.concatenate([lo, hi], axis=-1)


def _make_gather_kernel(ep):
    bg = 128
    bpt = ep // (_NW * bg)

    @functools.partial(
        pl.kernel,
        out_type=[
            jax.ShapeDtypeStruct((ep, 256), _F32),
            jax.ShapeDtypeStruct((ep, 128), _F32),
        ],
        mesh=_vector_mesh(),
        scratch_types=[
            pltpu.VMEM((bg,), _I32),
            pltpu.VMEM((bg,), _I32),
            pltpu.VMEM((bg,), _I32),
            pltpu.VMEM((bg,), _I32),
            pltpu.VMEM((bg, 256), _F32),
            pltpu.VMEM((bg, 256), _F32),
            pltpu.VMEM((bg, 128), _F32),
            pltpu.VMEM((bg, 128), _F32),
        ] + [pltpu.SemaphoreType.DMA] * 12,
    )
    def kg(c_hbm, r_hbm, rowi_hbm, colg_hbm, ec_hbm, er_hbm,
           ri0, ri1, cg0, cg1, cr0, cr1, rr0, rr1,
           s0, s1, s2, s3, s4, s5, s6, s7, s8, s9, s10, s11):
        cid = lax.axis_index("c")
        sid = lax.axis_index("s")
        wid = sid * _NC + cid
        ebase = wid * bpt * bg

        @pl.loop(0, bpt, step=2)
        def _(b):
            off0 = ebase + b * bg
            off1 = off0 + bg
            di0r = pltpu.async_copy(rowi_hbm.at[pl.ds(off0, bg)], ri0, s0)
            di0c = pltpu.async_copy(colg_hbm.at[pl.ds(off0, bg)], cg0, s1)
            di1r = pltpu.async_copy(rowi_hbm.at[pl.ds(off1, bg)], ri1, s2)
            di1c = pltpu.async_copy(colg_hbm.at[pl.ds(off1, bg)], cg1, s3)
            di0c.wait()
            g0c = pltpu.async_copy(c_hbm.at[cg0], cr0, s4)
            di0r.wait()
            g0r = pltpu.async_copy(r_hbm.at[ri0], rr0, s5)
            di1c.wait()
            g1c = pltpu.async_copy(c_hbm.at[cg1], cr1, s6)
            di1r.wait()
            g1r = pltpu.async_copy(r_hbm.at[ri1], rr1, s7)
            g0c.wait()
            o0c = pltpu.async_copy(cr0, ec_hbm.at[pl.ds(off0, bg)], s8)
            g0r.wait()
            o0r = pltpu.async_copy(rr0, er_hbm.at[pl.ds(off0, bg)], s9)
            g1c.wait()
            o1c = pltpu.async_copy(cr1, ec_hbm.at[pl.ds(off1, bg)], s10)
            g1r.wait()
            o1r = pltpu.async_copy(rr1, er_hbm.at[pl.ds(off1, bg)], s11)
            o0c.wait()
            o0r.wait()
            o1c.wait()
            o1r.wait()

    return kg


def _make_scatter_kernel(n_acc, n_rng, rng):
    rpt = rng // _NS
    dp = 128

    @functools.partial(
        pl.kernel,
        out_type=jax.ShapeDtypeStruct((_NC, n_acc, dp), _F32),
        mesh=_vector_mesh(),
        scratch_types=[
            pltpu.VMEM((16,), _I32),
            pltpu.VMEM((_B,), _I32),
            pltpu.VMEM((_B,), _I32),
            pltpu.VMEM((_B, dp), _F32),
            pltpu.VMEM((_B, dp), _F32),
            pltpu.VMEM((rpt, dp), _F32),
            pltpu.VMEM_SHARED((rng, dp), _F32),
        ] + [pltpu.SemaphoreType.DMA] * 6,
    )
    def ks(msg_hbm, coll_hbm, goff_hbm, out_hbm,
           offs_v, clA, clB, rowsA, rowsB, zbuf, acc_sh,
           s0, s1, s2, s3, s4, s5):
        cid = lax.axis_index("c")
        sid = lax.axis_index("s")
        wid = sid * _NC + cid
        zero16 = jnp.zeros((_L,), _F32)

        pltpu.sync_copy(goff_hbm, offs_v)
        offs = offs_v[...]

        @pl.loop(0, rpt)
        def _(i):
            for kk in range(dp // _L):
                zbuf[i, pl.ds(kk * _L, _L)] = zero16

        for r in range(n_rng):
            pltpu.sync_copy(zbuf, acc_sh.at[pl.ds(sid * rpt, rpt)])
            plsc.subcore_barrier()
            b0 = offs[r] // _B
            b1 = offs[r + 1] // _B

            @pl.loop(b0 + wid, b1, step=2 * _NW)
            def _(b):
                offA = b * _B
                offB = (b + _NW) * _B
                hb = b + _NW < b1
                dAc = pltpu.async_copy(coll_hbm.at[pl.ds(offA, _B)], clA, s0)
                dAm = pltpu.async_copy(msg_hbm.at[pl.ds(offA, _B)], rowsA, s1)

                @pl.when(hb)
                def _():
                    pltpu.async_copy(coll_hbm.at[pl.ds(offB, _B)], clB, s2)
                    pltpu.async_copy(msg_hbm.at[pl.ds(offB, _B)], rowsB, s3)

                dAc.wait()
                dAm.wait()
                sA = pltpu.async_copy(rowsA, acc_sh.at[clA], s4, add=True)

                @pl.when(hb)
                def _():
                    pltpu.make_async_copy(
                        coll_hbm.at[pl.ds(offB, _B)], clB, s2).wait()
                    pltpu.make_async_copy(
                        msg_hbm.at[pl.ds(offB, _B)], rowsB, s3).wait()
                    pltpu.async_copy(rowsB, acc_sh.at[clB], s5, add=True)

                sA.wait()

                @pl.when(hb)
                def _():
                    pltpu.make_async_copy(rowsB, acc_sh.at[clB], s5).wait()

            plsc.subcore_barrier()
            pltpu.sync_copy(
                acc_sh.at[pl.ds(sid * rpt, rpt)],
                out_hbm.at[cid, pl.ds(r * rng + sid * rpt, rpt)])
            plsc.subcore_barrier()

    return ks


_BLKE = 2048


def _tc_edge_coeff(ec, er):
    ep = ec.shape[0]

    def body(ec_ref, er_ref, msg_ref):
        xc = _unpack_pairs(ec_ref[:, :64], _BLKE)
        tt = _unpack_pairs(ec_ref[:, 64:128], _BLKE)
        gg = _unpack_pairs(ec_ref[:, 128:192], _BLKE)
        xj = _unpack_pairs(er_ref[:, :64], _BLKE)
        xm = _unpack_pairs(er_ref[:, 64:128], _BLKE)
        df = xj - xc
        dd = df * df
        ss = jnp.sum(dd, axis=-1, keepdims=True)
        tn = jnp.sum(tt * dd, axis=-1, keepdims=True)
        u = xc * xj
        wi = jnp.sum(gg * u, axis=-1, keepdims=True)
        ni = jnp.sum(gg * xc * xc, axis=-1, keepdims=True)
        nj = jnp.sum(gg * xj * xj, axis=-1, keepdims=True)
        tau = tn / jnp.maximum(ss, 1e-16)
        den = jnp.sqrt(jnp.maximum(ni * nj, 1e-30)) + 1e-8
        alpha = jax.nn.sigmoid(wi / den)
        msg_ref[...] = tau * alpha * xm

    return pl.pallas_call(
        body,
        grid=(ep // _BLKE,),
        in_specs=[
            pl.BlockSpec((_BLKE, 256), lambda i: (i, 0)),
            pl.BlockSpec((_BLKE, 128), lambda i: (i, 0)),
        ],
        out_specs=pl.BlockSpec((_BLKE, 128), lambda i: (i, 0)),
        out_shape=jax.ShapeDtypeStruct((ep, 128), _F32),
    )(ec, er)


def _ln(h, g, b):
    mu = jnp.mean(h, axis=-1, keepdims=True)
    var = jnp.mean((h - mu) * (h - mu), axis=-1, keepdims=True)
    return (h - mu) / jnp.sqrt(var + 1e-5) * g + b


_BLK = 1024


def _tc_encoder(xp, p):
    n_pad = xp.shape[0]

    def body(x_ref, w_ref, b_ref, g_ref, bt_ref, o_ref):
        h = jnp.dot(x_ref[...], w_ref[...],
                    preferred_element_type=_F32) + b_ref[...]
        o_ref[...] = jnp.maximum(_ln(h, g_ref[...], bt_ref[...]), 0.0)

    d_in = xp.shape[1]
    hid = p['enc_W'].shape[1]
    return pl.pallas_call(
        body,
        grid=(n_pad // _BLK,),
        in_specs=[
            pl.BlockSpec((_BLK, d_in), lambda i: (i, 0)),
            pl.BlockSpec((d_in, hid), lambda i: (0, 0)),
            pl.BlockSpec((hid,), lambda i: (0,)),
            pl.BlockSpec((hid,), lambda i: (0,)),
            pl.BlockSpec((hid,), lambda i: (0,)),
        ],
        out_specs=pl.BlockSpec((_BLK, hid), lambda i: (i, 0)),
        out_shape=jax.ShapeDtypeStruct((n_pad, hid), _F32),
    )(xp, p['enc_W'], p['enc_b'], p['enc_g'], p['enc_bt'])


def _tc_dense(h, ns2, deg2, l, dout):
    n_pad = h.shape[0]
    rw = 256

    def body(h_ref, ns_ref, deg_ref, mw1_ref, mb1_ref, mg_ref, mbt_ref,
             mw2_ref, mb2_ref, msgw_ref, selfw_ref, selfb_ref,
             c_ref, r_ref, so_ref):
        hh = h_ref[...]
        s = ns_ref[0] + ns_ref[1] + hh
        cnt = deg_ref[...] + 1.0
        x_nb = s / jnp.maximum(cnt, 1.0)[:, None]
        comb = jnp.concatenate([hh, x_nb], axis=-1)
        hm = jnp.dot(comb, mw1_ref[...],
                     preferred_element_type=_F32) + mb1_ref[...]
        hm = jnp.maximum(_ln(hm, mg_ref[...], mbt_ref[...]), 0.0)
        raw = jnp.dot(hm, mw2_ref[...],
                      preferred_element_type=_F32) + mb2_ref[...]
        z2 = 2.0 * raw
        sp = jnp.maximum(z2, 0.0) + jnp.log1p(jnp.exp(-jnp.abs(z2)))
        g = jnp.clip(sp * 0.5, 0.001, 10.0)
        t = (1.0 - g * g) / (1.0 + g * g)
        xm = jnp.dot(hh, msgw_ref[...], preferred_element_type=_F32)
        so = jnp.dot(hh, selfw_ref[...],
                     preferred_element_type=_F32) + selfb_ref[...]
        hp = _pack_pairs(hh, _BLK)
        tp = _pack_pairs(t, _BLK)
        gp = _pack_pairs(g, _BLK)
        zp = jnp.zeros((_BLK, 64), _F32)
        if dout == 128:
            xmp = _pack_pairs(xm, _BLK)
        else:
            xmp = _pack_pairs(
                jnp.concatenate([xm, jnp.zeros((_BLK, 128 - dout), _F32)],
                                axis=-1), _BLK)
        c_ref[...] = jnp.concatenate([hp, tp, gp, zp], axis=-1)
        r_ref[...] = jnp.concatenate([hp, xmp], axis=-1)
        so_ref[...] = so

    return pl.pallas_call(
        body,
        grid=(n_pad // _BLK,),
        in_specs=[
            pl.BlockSpec((_BLK, 128), lambda i: (i, 0)),
            pl.BlockSpec((2, _BLK, 128), lambda i: (0, i, 0)),
            pl.BlockSpec((_BLK,), lambda i: (i,)),
            pl.BlockSpec((256, 64), lambda i: (0, 0)),
            pl.BlockSpec((64,), lambda i: (0,)),
            pl.BlockSpec((64,), lambda i: (0,)),
            pl.BlockSpec((64,), lambda i: (0,)),
            pl.BlockSpec((64, 128), lambda i: (0, 0)),
            pl.BlockSpec((128,), lambda i: (0,)),
            pl.BlockSpec((128, dout), lambda i: (0, 0)),
            pl.BlockSpec((128, dout), lambda i: (0, 0)),
            pl.BlockSpec((dout,), lambda i: (0,)),
        ],
        out_specs=[
            pl.BlockSpec((_BLK, 256), lambda i: (i, 0)),
            pl.BlockSpec((_BLK, 128), lambda i: (i, 0)),
            pl.BlockSpec((_BLK, dout), lambda i: (i, 0)),
        ],
        out_shape=[
            jax.ShapeDtypeStruct((n_pad, 256), _F32),
            jax.ShapeDtypeStruct((n_pad, 128), _F32),
            jax.ShapeDtypeStruct((n_pad, dout), _F32),
        ],
    )(h, ns2, deg2, l['mW1'], l['mb1'], l['mg'], l['mbt'],
      l['mW2'], l['mb2'], l['msgW'], l['selfW'], l['selfb'])


def _tc_epilogue(agg2, so, l, last):
    n_pad = so.shape[0]
    dout = so.shape[1]

    def body(agg_ref, so_ref, g_ref, bt_ref, o_ref):
        agg = agg_ref[0][:, :dout] + agg_ref[1][:, :dout]
        o = _ln(agg + so_ref[...], g_ref[...], bt_ref[...])
        if last:
            m = jnp.max(o, axis=-1, keepdims=True)
            lse = jnp.log(jnp.sum(jnp.exp(o - m), axis=-1, keepdims=True)) + m
            o_ref[...] = o - lse
        else:
            o_ref[...] = jnp.maximum(o, 0.0)

    return pl.pallas_call(
        body,
        grid=(n_pad // _BLK,),
        in_specs=[
            pl.BlockSpec((2, _BLK, 128), lambda i: (0, i, 0)),
            pl.BlockSpec((_BLK, dout), lambda i: (i, 0)),
            pl.BlockSpec((dout,), lambda i: (0,)),
            pl.BlockSpec((dout,), lambda i: (0,)),
        ],
        out_specs=pl.BlockSpec((_BLK, dout), lambda i: (i, 0)),
        out_shape=jax.ShapeDtypeStruct((n_pad, dout), _F32),
    )(agg2, so, l['ng'], l['nbt'])


def kernel(x, params, edge_index):
    n, d_in = x.shape
    e = edge_index.shape[1]

    n_rng = 8
    rng = _NS * 80
    n_acc = n_rng * rng
    row = edge_index[0]
    col = edge_index[1]

    grp = col // rng
    pos = jnp.zeros((e,), _I32)
    goffs = []
    base = jnp.int32(0)
    pad_counts = []
    for r in range(n_rng):
        m = grp == r
        ranks = jnp.cumsum(m.astype(_I32))
        size = ranks[-1]
        goffs.append(base)
        pos = jnp.where(m, base + ranks - 1, pos)
        padded = ((size + _B - 1) // _B) * _B
        pad_counts.append(padded - size)
        base = base + padded
    goffs.append(base)
    bpt = (e + n_rng * _B + _NW * _B - 1) // (_NW * _B)
    ep = bpt * _NW * _B
    rowp = jnp.full((ep,), n, _I32).at[pos].set(row)
    colp = jnp.zeros((ep,), _I32).at[pos].set(col - grp * rng)
    colg = jnp.full((ep,), n, _I32).at[pos].set(col)
    goff = jnp.zeros((16,), _I32).at[:n_rng + 1].set(jnp.stack(goffs))
    padvec = jnp.zeros((n_acc,), _F32).at[
        jnp.arange(n_rng) * rng].set(jnp.stack(pad_counts).astype(_F32))

    xp = jnp.pad(x, ((0, n_acc - n), (0, 0)))

    deg_out = _make_deg_kernel(n_acc, n_rng, rng)(colp, goff)
    deg = deg_out[0, :, 0] + deg_out[1, :, 0] - padvec
    h = _tc_encoder(xp, params)

    nsum = _make_nsum_kernel(n_acc, 128, n_rng, rng)
    gatherk = _make_gather_kernel(ep)
    scatterk = _make_scatter_kernel(n_acc, n_rng, rng)
    n_layers = len(params['layers'])
    for i, l in enumerate(params['layers']):
        dout = l['msgW'].shape[1]
        ns2 = nsum(h, rowp, colp, goff)
        c_tbl, r_tbl, so = _tc_dense(h, ns2, deg, l, dout)
        ec, er = gatherk(c_tbl, r_tbl, rowp, colg)
        msg = _tc_edge_coeff(ec, er)
        agg2 = scatterk(msg, colp, goff)
        h = _tc_epilogue(agg2, so, l, last=(i == n_layers - 1))

    return h[:n]


# speedup vs baseline: 2.2065x; 1.0026x over previous
import dataclasses
import functools

import jax
import jax.numpy as jnp
from jax import lax
from jax.experimental import pallas as pl
from jax.experimental.pallas import tpu as pltpu
from jax.experimental.pallas import tpu_sc as plsc

_NC = 2
_NS = 16
_NW = _NC * _NS
_L = 16
_B = 128

_F32 = jnp.float32
_I32 = jnp.int32


def _sc_compiler_params():
    cp = pltpu.CompilerParams()
    if "needs_layout_passes" in pltpu.CompilerParams.__dataclass_fields__:
        cp = dataclasses.replace(cp, needs_layout_passes=False)
    return cp


def _vector_mesh():
    return plsc.VectorSubcoreMesh(core_axis_name="c", subcore_axis_name="s",
                                  num_cores=_NC, num_subcores=_NS)


def _make_deg_kernel(n_acc, n_rng, rng):
    rpt = rng // _NS

    @functools.partial(
        pl.kernel,
        out_type=jax.ShapeDtypeStruct((_NC, n_acc, 128), _F32),
        mesh=_vector_mesh(),
        scratch_types=[
            pltpu.VMEM((16,), _I32),
            pltpu.VMEM((_B,), _I32),
            pltpu.VMEM((_B, 128), _F32),
            pltpu.VMEM((rpt, 128), _F32),
            pltpu.VMEM_SHARED((rng, 128), _F32),
        ],
    )
    def kd(coli_hbm, goff_hbm, out_hbm, offs_v, ci_v, ones_v, zbuf, acc_sh):
        cid = lax.axis_index("c")
        sid = lax.axis_index("s")
        wid = sid * _NC + cid
        zero16 = jnp.zeros((_L,), _F32)
        e0 = jnp.where(lax.iota(_I32, _L) == 0, 1.0, 0.0).astype(_F32)

        pltpu.sync_copy(goff_hbm, offs_v)
        offs = offs_v[...]

        @pl.loop(0, rpt)
        def _(i):
            for kk in range(128 // _L):
                zbuf[i, pl.ds(kk * _L, _L)] = zero16

        @pl.loop(0, _B)
        def _(i):
            ones_v[i, pl.ds(0, _L)] = e0
            for kk in range(1, 128 // _L):
                ones_v[i, pl.ds(kk * _L, _L)] = zero16

        for r in range(n_rng):
            pltpu.sync_copy(zbuf, acc_sh.at[pl.ds(sid * rpt, rpt)])
            plsc.subcore_barrier()
            b0 = offs[r] // _B
            b1 = offs[r + 1] // _B

            @pl.loop(b0 + wid, b1, step=_NW)
            def _(b):
                pltpu.sync_copy(coli_hbm.at[pl.ds(b * _B, _B)], ci_v)
                pltpu.sync_copy(ones_v, acc_sh.at[ci_v], add=True)

            plsc.subcore_barrier()
            pltpu.sync_copy(
                acc_sh.at[pl.ds(sid * rpt, rpt)],
                out_hbm.at[cid, pl.ds(r * rng + sid * rpt, rpt)])
            plsc.subcore_barrier()

    return kd


def _make_nsum_kernel(n_acc, width, n_rng, rng):
    rpt = rng // _NS

    @functools.partial(
        pl.kernel,
        out_type=jax.ShapeDtypeStruct((_NC, n_acc, width), _F32),
        mesh=_vector_mesh(),
        scratch_types=[
            pltpu.VMEM((16,), _I32),
            pltpu.VMEM((_B,), _I32),
            pltpu.VMEM((_B,), _I32),
            pltpu.VMEM((_B,), _I32),
            pltpu.VMEM((_B,), _I32),
            pltpu.VMEM((_B, width), _F32),
            pltpu.VMEM((_B, width), _F32),
            pltpu.VMEM((rpt, width), _F32),
            pltpu.VMEM_SHARED((rng, width), _F32),
        ] + [pltpu.SemaphoreType.DMA] * 8,
    )
    def ka(tbl_hbm, rowi_hbm, coli_hbm, goff_hbm, out_hbm,
           offs_v, riA, ciA, riB, ciB, rowsA, rowsB, zbuf, acc_sh,
           s0, s1, s2, s3, s4, s5, s6, s7):
        cid = lax.axis_index("c")
        sid = lax.axis_index("s")
        wid = sid * _NC + cid
        zero16 = jnp.zeros((_L,), _F32)

        pltpu.sync_copy(goff_hbm, offs_v)
        offs = offs_v[...]

        @pl.loop(0, rpt)
        def _(i):
            for kk in range(width // _L):
                zbuf[i, pl.ds(kk * _L, _L)] = zero16

        for r in range(n_rng):
            pltpu.sync_copy(zbuf, acc_sh.at[pl.ds(sid * rpt, rpt)])
            plsc.subcore_barrier()
            b0 = offs[r] // _B
            b1 = offs[r + 1] // _B

            @pl.loop(b0 + wid, b1, step=2 * _NW)
            def _(b):
                offA = b * _B
                offB = (b + _NW) * _B
                hb = b + _NW < b1
                dAr = pltpu.async_copy(rowi_hbm.at[pl.ds(offA, _B)], riA, s0)
                dAc = pltpu.async_copy(coli_hbm.at[pl.ds(offA, _B)], ciA, s1)

                @pl.when(hb)
                def _():
                    pltpu.async_copy(rowi_hbm.at[pl.ds(offB, _B)], riB, s2)
                    pltpu.async_copy(coli_hbm.at[pl.ds(offB, _B)], ciB, s3)

                dAr.wait()
                gA = pltpu.async_copy(tbl_hbm.at[riA], rowsA, s4)

                @pl.when(hb)
                def _():
                    pltpu.make_async_copy(
                        rowi_hbm.at[pl.ds(offB, _B)], riB, s2).wait()
                    pltpu.async_copy(tbl_hbm.at[riB], rowsB, s5)

                gA.wait()
                dAc.wait()
                sA = pltpu.async_copy(rowsA, acc_sh.at[ciA], add=True, sem=s6)

                @pl.when(hb)
                def _():
                    pltpu.make_async_copy(tbl_hbm.at[riB], rowsB, s5).wait()
                    pltpu.make_async_copy(
                        coli_hbm.at[pl.ds(offB, _B)], ciB, s3).wait()
                    pltpu.async_copy(rowsB, acc_sh.at[ciB], add=True, sem=s7)

                sA.wait()

                @pl.when(hb)
                def _():
                    pltpu.make_async_copy(rowsB, acc_sh.at[ciB], s7).wait()

            plsc.subcore_barrier()
            pltpu.sync_copy(
                acc_sh.at[pl.ds(sid * rpt, rpt)],
                out_hbm.at[cid, pl.ds(r * rng + sid * rpt, rpt)])
            plsc.subcore_barrier()

    return ka


def _pack_pairs(a, nrows):
    ia = lax.bitcast_convert_type(a, _I32)
    lo = lax.shift_right_logical(ia[:, :64] + jnp.int32(0x8000), 16)
    hi = lax.shift_right_logical(ia[:, 64:] + jnp.int32(0x8000), 16)
    word = jnp.bitwise_or(lax.shift_left(hi, 16), lo)
    return lax.bitcast_convert_type(word, _F32)


def _unpack_pairs(w, nrows):
    iw = lax.bitcast_convert_type(w, _I32)
    lo = lax.bitcast_convert_type(lax.shift_left(iw, 16), _F32)
    hi = lax.bitcast_convert_type(
        jnp.bitwise_and(iw, jnp.int32(-65536)), _F32)
    return jnp.concatenate([lo, hi], axis=-1)


def _make_gather_kernel(ep):
    bg = 128
    bpt = ep // (_NW * bg)

    @functools.partial(
        pl.kernel,
        out_type=[
            jax.ShapeDtypeStruct((ep, 256), _F32),
            jax.ShapeDtypeStruct((ep, 128), _F32),
        ],
        mesh=_vector_mesh(),
        scratch_types=[
            pltpu.VMEM((bg,), _I32),
            pltpu.VMEM((bg,), _I32),
            pltpu.VMEM((bg,), _I32),
            pltpu.VMEM((bg,), _I32),
            pltpu.VMEM((bg, 256), _F32),
            pltpu.VMEM((bg, 256), _F32),
            pltpu.VMEM((bg, 128), _F32),
            pltpu.VMEM((bg, 128), _F32),
        ] + [pltpu.SemaphoreType.DMA] * 12,
    )
    def kg(c_hbm, r_hbm, rowi_hbm, colg_hbm, ec_hbm, er_hbm,
           ri0, ri1, cg0, cg1, cr0, cr1, rr0, rr1,
           s0, s1, s2, s3, s4, s5, s6, s7, s8, s9, s10, s11):
        cid = lax.axis_index("c")
        sid = lax.axis_index("s")
        wid = sid * _NC + cid
        ebase = wid * bpt * bg

        @pl.loop(0, bpt, step=2)
        def _(b):
            off0 = ebase + b * bg
            off1 = off0 + bg
            di0r = pltpu.async_copy(rowi_hbm.at[pl.ds(off0, bg)], ri0, s0)
            di0c = pltpu.async_copy(colg_hbm.at[pl.ds(off0, bg)], cg0, s1)
            di1r = pltpu.async_copy(rowi_hbm.at[pl.ds(off1, bg)], ri1, s2)
            di1c = pltpu.async_copy(colg_hbm.at[pl.ds(off1, bg)], cg1, s3)

            @pl.when(b > 0)
            def _():
                pltpu.make_async_copy(cr0, ec_hbm.at[pl.ds(0, bg)], s8).wait()
                pltpu.make_async_copy(rr0, er_hbm.at[pl.ds(0, bg)], s9).wait()
                pltpu.make_async_copy(cr1, ec_hbm.at[pl.ds(0, bg)], s10).wait()
                pltpu.make_async_copy(rr1, er_hbm.at[pl.ds(0, bg)], s11).wait()

            di0c.wait()
            g0c = pltpu.async_copy(c_hbm.at[cg0], cr0, s4)
            di0r.wait()
            g0r = pltpu.async_copy(r_hbm.at[ri0], rr0, s5)
            di1c.wait()
            g1c = pltpu.async_copy(c_hbm.at[cg1], cr1, s6)
            di1r.wait()
            g1r = pltpu.async_copy(r_hbm.at[ri1], rr1, s7)
            g0c.wait()
            pltpu.async_copy(cr0, ec_hbm.at[pl.ds(off0, bg)], s8)
            g0r.wait()
            pltpu.async_copy(rr0, er_hbm.at[pl.ds(off0, bg)], s9)
            g1c.wait()
            pltpu.async_copy(cr1, ec_hbm.at[pl.ds(off1, bg)], s10)
            g1r.wait()
            pltpu.async_copy(rr1, er_hbm.at[pl.ds(off1, bg)], s11)

        pltpu.make_async_copy(cr0, ec_hbm.at[pl.ds(0, bg)], s8).wait()
        pltpu.make_async_copy(rr0, er_hbm.at[pl.ds(0, bg)], s9).wait()
        pltpu.make_async_copy(cr1, ec_hbm.at[pl.ds(0, bg)], s10).wait()
        pltpu.make_async_copy(rr1, er_hbm.at[pl.ds(0, bg)], s11).wait()

    return kg


def _make_scatter_kernel(n_acc, n_rng, rng):
    rpt = rng // _NS
    dp = 128

    @functools.partial(
        pl.kernel,
        out_type=jax.ShapeDtypeStruct((_NC, n_acc, dp), _F32),
        mesh=_vector_mesh(),
        scratch_types=[
            pltpu.VMEM((16,), _I32),
            pltpu.VMEM((_B,), _I32),
            pltpu.VMEM((_B,), _I32),
            pltpu.VMEM((_B, dp), _F32),
            pltpu.VMEM((_B, dp), _F32),
            pltpu.VMEM((rpt, dp), _F32),
            pltpu.VMEM_SHARED((rng, dp), _F32),
        ] + [pltpu.SemaphoreType.DMA] * 6,
    )
    def ks(msg_hbm, coll_hbm, goff_hbm, out_hbm,
           offs_v, clA, clB, rowsA, rowsB, zbuf, acc_sh,
           s0, s1, s2, s3, s4, s5):
        cid = lax.axis_index("c")
        sid = lax.axis_index("s")
        wid = sid * _NC + cid
        zero16 = jnp.zeros((_L,), _F32)

        pltpu.sync_copy(goff_hbm, offs_v)
        offs = offs_v[...]

        @pl.loop(0, rpt)
        def _(i):
            for kk in range(dp // _L):
                zbuf[i, pl.ds(kk * _L, _L)] = zero16

        for r in range(n_rng):
            pltpu.sync_copy(zbuf, acc_sh.at[pl.ds(sid * rpt, rpt)])
            plsc.subcore_barrier()
            b0 = offs[r] // _B
            b1 = offs[r + 1] // _B

            @pl.loop(b0 + wid, b1, step=2 * _NW)
            def _(b):
                offA = b * _B
                offB = (b + _NW) * _B
                hb = b + _NW < b1
                dAc = pltpu.async_copy(coll_hbm.at[pl.ds(offA, _B)], clA, s0)
                dAm = pltpu.async_copy(msg_hbm.at[pl.ds(offA, _B)], rowsA, s1)

                @pl.when(hb)
                def _():
                    pltpu.async_copy(coll_hbm.at[pl.ds(offB, _B)], clB, s2)
                    pltpu.async_copy(msg_hbm.at[pl.ds(offB, _B)], rowsB, s3)

                dAc.wait()
                dAm.wait()
                sA = pltpu.async_copy(rowsA, acc_sh.at[clA], s4, add=True)

                @pl.when(hb)
                def _():
                    pltpu.make_async_copy(
                        coll_hbm.at[pl.ds(offB, _B)], clB, s2).wait()
                    pltpu.make_async_copy(
                        msg_hbm.at[pl.ds(offB, _B)], rowsB, s3).wait()
                    pltpu.async_copy(rowsB, acc_sh.at[clB], s5, add=True)

                sA.wait()

                @pl.when(hb)
                def _():
                    pltpu.make_async_copy(rowsB, acc_sh.at[clB], s5).wait()

            plsc.subcore_barrier()
            pltpu.sync_copy(
                acc_sh.at[pl.ds(sid * rpt, rpt)],
                out_hbm.at[cid, pl.ds(r * rng + sid * rpt, rpt)])
            plsc.subcore_barrier()

    return ks


_BLKE = 2048


def _tc_edge_coeff(ec, er):
    ep = ec.shape[0]

    def body(ec_ref, er_ref, msg_ref):
        xc = _unpack_pairs(ec_ref[:, :64], _BLKE)
        tt = _unpack_pairs(ec_ref[:, 64:128], _BLKE)
        gg = _unpack_pairs(ec_ref[:, 128:192], _BLKE)
        xj = _unpack_pairs(er_ref[:, :64], _BLKE)
        xm = _unpack_pairs(er_ref[:, 64:128], _BLKE)
        df = xj - xc
        dd = df * df
        ss = jnp.sum(dd, axis=-1, keepdims=True)
        tn = jnp.sum(tt * dd, axis=-1, keepdims=True)
        u = xc * xj
        wi = jnp.sum(gg * u, axis=-1, keepdims=True)
        ni = jnp.sum(gg * xc * xc, axis=-1, keepdims=True)
        nj = jnp.sum(gg * xj * xj, axis=-1, keepdims=True)
        tau = tn / jnp.maximum(ss, 1e-16)
        den = jnp.sqrt(jnp.maximum(ni * nj, 1e-30)) + 1e-8
        alpha = jax.nn.sigmoid(wi / den)
        msg_ref[...] = tau * alpha * xm

    return pl.pallas_call(
        body,
        grid=(ep // _BLKE,),
        in_specs=[
            pl.BlockSpec((_BLKE, 256), lambda i: (i, 0)),
            pl.BlockSpec((_BLKE, 128), lambda i: (i, 0)),
        ],
        out_specs=pl.BlockSpec((_BLKE, 128), lambda i: (i, 0)),
        out_shape=jax.ShapeDtypeStruct((ep, 128), _F32),
    )(ec, er)


def _ln(h, g, b):
    mu = jnp.mean(h, axis=-1, keepdims=True)
    var = jnp.mean((h - mu) * (h - mu), axis=-1, keepdims=True)
    return (h - mu) / jnp.sqrt(var + 1e-5) * g + b


_BLK = 1024


def _tc_encoder(xp, p):
    n_pad = xp.shape[0]

    def body(x_ref, w_ref, b_ref, g_ref, bt_ref, o_ref):
        h = jnp.dot(x_ref[...], w_ref[...],
                    preferred_element_type=_F32) + b_ref[...]
        o_ref[...] = jnp.maximum(_ln(h, g_ref[...], bt_ref[...]), 0.0)

    d_in = xp.shape[1]
    hid = p['enc_W'].shape[1]
    return pl.pallas_call(
        body,
        grid=(n_pad // _BLK,),
        in_specs=[
            pl.BlockSpec((_BLK, d_in), lambda i: (i, 0)),
            pl.BlockSpec((d_in, hid), lambda i: (0, 0)),
            pl.BlockSpec((hid,), lambda i: (0,)),
            pl.BlockSpec((hid,), lambda i: (0,)),
            pl.BlockSpec((hid,), lambda i: (0,)),
        ],
        out_specs=pl.BlockSpec((_BLK, hid), lambda i: (i, 0)),
        out_shape=jax.ShapeDtypeStruct((n_pad, hid), _F32),
    )(xp, p['enc_W'], p['enc_b'], p['enc_g'], p['enc_bt'])


def _tc_dense(h, ns2, deg2, l, dout):
    n_pad = h.shape[0]
    rw = 256

    def body(h_ref, ns_ref, deg_ref, mw1_ref, mb1_ref, mg_ref, mbt_ref,
             mw2_ref, mb2_ref, msgw_ref, selfw_ref, selfb_ref,
             c_ref, r_ref, so_ref):
        hh = h_ref[...]
        s = ns_ref[0] + ns_ref[1] + hh
        cnt = deg_ref[...] + 1.0
        x_nb = s / jnp.maximum(cnt, 1.0)[:, None]
        comb = jnp.concatenate([hh, x_nb], axis=-1)
        hm = jnp.dot(comb, mw1_ref[...],
                     preferred_element_type=_F32) + mb1_ref[...]
        hm = jnp.maximum(_ln(hm, mg_ref[...], mbt_ref[...]), 0.0)
        raw = jnp.dot(hm, mw2_ref[...],
                      preferred_element_type=_F32) + mb2_ref[...]
        z2 = 2.0 * raw
        sp = jnp.maximum(z2, 0.0) + jnp.log1p(jnp.exp(-jnp.abs(z2)))
        g = jnp.clip(sp * 0.5, 0.001, 10.0)
        t = (1.0 - g * g) / (1.0 + g * g)
        xm = jnp.dot(hh, msgw_ref[...], preferred_element_type=_F32)
        so = jnp.dot(hh, selfw_ref[...],
                     preferred_element_type=_F32) + selfb_ref[...]
        hp = _pack_pairs(hh, _BLK)
        tp = _pack_pairs(t, _BLK)
        gp = _pack_pairs(g, _BLK)
        zp = jnp.zeros((_BLK, 64), _F32)
        if dout == 128:
            xmp = _pack_pairs(xm, _BLK)
        else:
            xmp = _pack_pairs(
                jnp.concatenate([xm, jnp.zeros((_BLK, 128 - dout), _F32)],
                                axis=-1), _BLK)
        c_ref[...] = jnp.concatenate([hp, tp, gp, zp], axis=-1)
        r_ref[...] = jnp.concatenate([hp, xmp], axis=-1)
        so_ref[...] = so

    return pl.pallas_call(
        body,
        grid=(n_pad // _BLK,),
        in_specs=[
            pl.BlockSpec((_BLK, 128), lambda i: (i, 0)),
            pl.BlockSpec((2, _BLK, 128), lambda i: (0, i, 0)),
            pl.BlockSpec((_BLK,), lambda i: (i,)),
            pl.BlockSpec((256, 64), lambda i: (0, 0)),
            pl.BlockSpec((64,), lambda i: (0,)),
            pl.BlockSpec((64,), lambda i: (0,)),
            pl.BlockSpec((64,), lambda i: (0,)),
            pl.BlockSpec((64, 128), lambda i: (0, 0)),
            pl.BlockSpec((128,), lambda i: (0,)),
            pl.BlockSpec((128, dout), lambda i: (0, 0)),
            pl.BlockSpec((128, dout), lambda i: (0, 0)),
            pl.BlockSpec((dout,), lambda i: (0,)),
        ],
        out_specs=[
            pl.BlockSpec((_BLK, 256), lambda i: (i, 0)),
            pl.BlockSpec((_BLK, 128), lambda i: (i, 0)),
            pl.BlockSpec((_BLK, dout), lambda i: (i, 0)),
        ],
        out_shape=[
            jax.ShapeDtypeStruct((n_pad, 256), _F32),
            jax.ShapeDtypeStruct((n_pad, 128), _F32),
            jax.ShapeDtypeStruct((n_pad, dout), _F32),
        ],
    )(h, ns2, deg2, l['mW1'], l['mb1'], l['mg'], l['mbt'],
      l['mW2'], l['mb2'], l['msgW'], l['selfW'], l['selfb'])


def _tc_epilogue(agg2, so, l, last):
    n_pad = so.shape[0]
    dout = so.shape[1]

    def body(agg_ref, so_ref, g_ref, bt_ref, o_ref):
        agg = agg_ref[0][:, :dout] + agg_ref[1][:, :dout]
        o = _ln(agg + so_ref[...], g_ref[...], bt_ref[...])
        if last:
            m = jnp.max(o, axis=-1, keepdims=True)
            lse = jnp.log(jnp.sum(jnp.exp(o - m), axis=-1, keepdims=True)) + m
            o_ref[...] = o - lse
        else:
            o_ref[...] = jnp.maximum(o, 0.0)

    return pl.pallas_call(
        body,
        grid=(n_pad // _BLK,),
        in_specs=[
            pl.BlockSpec((2, _BLK, 128), lambda i: (0, i, 0)),
            pl.BlockSpec((_BLK, dout), lambda i: (i, 0)),
            pl.BlockSpec((dout,), lambda i: (0,)),
            pl.BlockSpec((dout,), lambda i: (0,)),
        ],
        out_specs=pl.BlockSpec((_BLK, dout), lambda i: (i, 0)),
        out_shape=jax.ShapeDtypeStruct((n_pad, dout), _F32),
    )(agg2, so, l['ng'], l['nbt'])


def kernel(x, params, edge_index):
    n, d_in = x.shape
    e = edge_index.shape[1]

    n_rng = 8
    rng = _NS * 80
    n_acc = n_rng * rng
    row = edge_index[0]
    col = edge_index[1]

    grp = col // rng
    pos = jnp.zeros((e,), _I32)
    goffs = []
    base = jnp.int32(0)
    pad_counts = []
    for r in range(n_rng):
        m = grp == r
        ranks = jnp.cumsum(m.astype(_I32))
        size = ranks[-1]
        goffs.append(base)
        pos = jnp.where(m, base + ranks - 1, pos)
        padded = ((size + _B - 1) // _B) * _B
        pad_counts.append(padded - size)
        base = base + padded
    goffs.append(base)
    bpt = (e + n_rng * _B + _NW * _B - 1) // (_NW * _B)
    ep = bpt * _NW * _B
    rowp = jnp.full((ep,), n, _I32).at[pos].set(row)
    colp = jnp.zeros((ep,), _I32).at[pos].set(col - grp * rng)
    colg = jnp.full((ep,), n, _I32).at[pos].set(col)
    goff = jnp.zeros((16,), _I32).at[:n_rng + 1].set(jnp.stack(goffs))
    padvec = jnp.zeros((n_acc,), _F32).at[
        jnp.arange(n_rng) * rng].set(jnp.stack(pad_counts).astype(_F32))

    xp = jnp.pad(x, ((0, n_acc - n), (0, 0)))

    deg_out = _make_deg_kernel(n_acc, n_rng, rng)(colp, goff)
    deg = deg_out[0, :, 0] + deg_out[1, :, 0] - padvec
    h = _tc_encoder(xp, params)

    nsum = _make_nsum_kernel(n_acc, 128, n_rng, rng)
    gatherk = _make_gather_kernel(ep)
    scatterk = _make_scatter_kernel(n_acc, n_rng, rng)
    n_layers = len(params['layers'])
    for i, l in enumerate(params['layers']):
        dout = l['msgW'].shape[1]
        ns2 = nsum(h, rowp, colp, goff)
        c_tbl, r_tbl, so = _tc_dense(h, ns2, deg, l, dout)
        ec, er = gatherk(c_tbl, r_tbl, rowp, colg)
        msg = _tc_edge_coeff(ec, er)
        agg2 = scatterk(msg, colp, goff)
        h = _tc_epilogue(agg2, so, l, last=(i == n_layers - 1))

    return h[:n]


# speedup vs baseline: 2.2066x; 1.0000x over previous
import functools

import jax
import jax.numpy as jnp
from jax import lax
from jax.experimental import pallas as pl
from jax.experimental.pallas import tpu as pltpu
from jax.experimental.pallas import tpu_sc as plsc

_NC = 2
_NS = 16
_NW = _NC * _NS
_L = 16
_B = 128

_F32 = jnp.float32
_I32 = jnp.int32


def _vector_mesh():
    return plsc.VectorSubcoreMesh(core_axis_name="c", subcore_axis_name="s",
                                  num_cores=_NC, num_subcores=_NS)


def _make_deg_kernel(n_acc, n_rng, rng):
    rpt = rng // _NS

    @functools.partial(
        pl.kernel,
        out_type=jax.ShapeDtypeStruct((_NC, n_acc, 128), _F32),
        mesh=_vector_mesh(),
        scratch_types=[
            pltpu.VMEM((16,), _I32),
            pltpu.VMEM((_B,), _I32),
            pltpu.VMEM((_B, 128), _F32),
            pltpu.VMEM((rpt, 128), _F32),
            pltpu.VMEM_SHARED((rng, 128), _F32),
        ],
    )
    def kd(coli_hbm, goff_hbm, out_hbm, offs_v, ci_v, ones_v, zbuf, acc_sh):
        cid = lax.axis_index("c")
        sid = lax.axis_index("s")
        wid = sid * _NC + cid
        zero16 = jnp.zeros((_L,), _F32)
        e0 = jnp.where(lax.iota(_I32, _L) == 0, 1.0, 0.0).astype(_F32)

        pltpu.sync_copy(goff_hbm, offs_v)
        offs = offs_v[...]

        @pl.loop(0, rpt)
        def _(i):
            for kk in range(128 // _L):
                zbuf[i, pl.ds(kk * _L, _L)] = zero16

        @pl.loop(0, _B)
        def _(i):
            ones_v[i, pl.ds(0, _L)] = e0
            for kk in range(1, 128 // _L):
                ones_v[i, pl.ds(kk * _L, _L)] = zero16

        for r in range(n_rng):
            pltpu.sync_copy(zbuf, acc_sh.at[pl.ds(sid * rpt, rpt)])
            plsc.subcore_barrier()
            b0 = offs[r] // _B
            b1 = offs[r + 1] // _B

            @pl.loop(b0 + wid, b1, step=_NW)
            def _(b):
                pltpu.sync_copy(coli_hbm.at[pl.ds(b * _B, _B)], ci_v)
                pltpu.sync_copy(ones_v, acc_sh.at[ci_v], add=True)

            plsc.subcore_barrier()
            pltpu.sync_copy(
                acc_sh.at[pl.ds(sid * rpt, rpt)],
                out_hbm.at[cid, pl.ds(r * rng + sid * rpt, rpt)])
            plsc.subcore_barrier()

    return kd


def _make_nsum_kernel(n_acc, width, n_rng, rng):
    rpt = rng // _NS

    @functools.partial(
        pl.kernel,
        out_type=jax.ShapeDtypeStruct((_NC, n_acc, width), _F32),
        mesh=_vector_mesh(),
        scratch_types=[
            pltpu.VMEM((16,), _I32),
            pltpu.VMEM((_B,), _I32),
            pltpu.VMEM((_B,), _I32),
            pltpu.VMEM((_B,), _I32),
            pltpu.VMEM((_B,), _I32),
            pltpu.VMEM((_B, width), _F32),
            pltpu.VMEM((_B, width), _F32),
            pltpu.VMEM((rpt, width), _F32),
            pltpu.VMEM_SHARED((rng, width), _F32),
        ] + [pltpu.SemaphoreType.DMA] * 8,
    )
    def ka(tbl_hbm, rowi_hbm, coli_hbm, goff_hbm, out_hbm,
           offs_v, riA, ciA, riB, ciB, rowsA, rowsB, zbuf, acc_sh,
           s0, s1, s2, s3, s4, s5, s6, s7):
        cid = lax.axis_index("c")
        sid = lax.axis_index("s")
        wid = sid * _NC + cid
        zero16 = jnp.zeros((_L,), _F32)

        pltpu.sync_copy(goff_hbm, offs_v)
        offs = offs_v[...]

        @pl.loop(0, rpt)
        def _(i):
            for kk in range(width // _L):
                zbuf[i, pl.ds(kk * _L, _L)] = zero16

        for r in range(n_rng):
            pltpu.sync_copy(zbuf, acc_sh.at[pl.ds(sid * rpt, rpt)])
            plsc.subcore_barrier()
            b0 = offs[r] // _B
            b1 = offs[r + 1] // _B

            @pl.loop(b0 + wid, b1, step=2 * _NW)
            def _(b):
                offA = b * _B
                offB = (b + _NW) * _B
                hb = b + _NW < b1
                dAr = pltpu.async_copy(rowi_hbm.at[pl.ds(offA, _B)], riA, s0)
                dAc = pltpu.async_copy(coli_hbm.at[pl.ds(offA, _B)], ciA, s1)

                @pl.when(hb)
                def _():
                    pltpu.async_copy(rowi_hbm.at[pl.ds(offB, _B)], riB, s2)
                    pltpu.async_copy(coli_hbm.at[pl.ds(offB, _B)], ciB, s3)

                dAr.wait()
                gA = pltpu.async_copy(tbl_hbm.at[riA], rowsA, s4)

                @pl.when(hb)
                def _():
                    pltpu.make_async_copy(
                        rowi_hbm.at[pl.ds(offB, _B)], riB, s2).wait()
                    pltpu.async_copy(tbl_hbm.at[riB], rowsB, s5)

                gA.wait()
                dAc.wait()
                sA = pltpu.async_copy(rowsA, acc_sh.at[ciA], add=True, sem=s6)

                @pl.when(hb)
                def _():
                    pltpu.make_async_copy(tbl_hbm.at[riB], rowsB, s5).wait()
                    pltpu.make_async_copy(
                        coli_hbm.at[pl.ds(offB, _B)], ciB, s3).wait()
                    pltpu.async_copy(rowsB, acc_sh.at[ciB], add=True, sem=s7)

                sA.wait()

                @pl.when(hb)
                def _():
                    pltpu.make_async_copy(rowsB, acc_sh.at[ciB], s7).wait()

            plsc.subcore_barrier()
            pltpu.sync_copy(
                acc_sh.at[pl.ds(sid * rpt, rpt)],
                out_hbm.at[cid, pl.ds(r * rng + sid * rpt, rpt)])
            plsc.subcore_barrier()

    return ka


def _pack_pairs(a, nrows):
    ia = lax.bitcast_convert_type(a, _I32)
    lo = lax.shift_right_logical(ia[:, :64] + jnp.int32(0x8000), 16)
    hi = lax.shift_right_logical(ia[:, 64:] + jnp.int32(0x8000), 16)
    word = jnp.bitwise_or(lax.shift_left(hi, 16), lo)
    return lax.bitcast_convert_type(word, _F32)


def _unpack_pairs(w, nrows):
    iw = lax.bitcast_convert_type(w, _I32)
    lo = lax.bitcast_convert_type(lax.shift_left(iw, 16), _F32)
    hi = lax.bitcast_convert_type(
        jnp.bitwise_and(iw, jnp.int32(-65536)), _F32)
    return jnp.concatenate([lo, hi], axis=-1)


def _make_gather_kernel(ep):
    bg = 128
    bpt = ep // (_NW * bg)

    @functools.partial(
        pl.kernel,
        out_type=[
            jax.ShapeDtypeStruct((ep, 256), _F32),
            jax.ShapeDtypeStruct((ep, 128), _F32),
        ],
        mesh=_vector_mesh(),
        scratch_types=[
            pltpu.VMEM((bg,), _I32),
            pltpu.VMEM((bg,), _I32),
            pltpu.VMEM((bg,), _I32),
            pltpu.VMEM((bg,), _I32),
            pltpu.VMEM((bg, 256), _F32),
            pltpu.VMEM((bg, 256), _F32),
            pltpu.VMEM((bg, 128), _F32),
            pltpu.VMEM((bg, 128), _F32),
        ] + [pltpu.SemaphoreType.DMA] * 12,
    )
    def kg(c_hbm, r_hbm, rowi_hbm, colg_hbm, ec_hbm, er_hbm,
           ri0, ri1, cg0, cg1, cr0, cr1, rr0, rr1,
           s0, s1, s2, s3, s4, s5, s6, s7, s8, s9, s10, s11):
        cid = lax.axis_index("c")
        sid = lax.axis_index("s")
        wid = sid * _NC + cid
        ebase = wid * bpt * bg

        @pl.loop(0, bpt, step=2)
        def _(b):
            off0 = ebase + b * bg
            off1 = off0 + bg
            di0r = pltpu.async_copy(rowi_hbm.at[pl.ds(off0, bg)], ri0, s0)
            di0c = pltpu.async_copy(colg_hbm.at[pl.ds(off0, bg)], cg0, s1)
            di1r = pltpu.async_copy(rowi_hbm.at[pl.ds(off1, bg)], ri1, s2)
            di1c = pltpu.async_copy(colg_hbm.at[pl.ds(off1, bg)], cg1, s3)

            @pl.when(b > 0)
            def _():
                pltpu.make_async_copy(cr0, ec_hbm.at[pl.ds(0, bg)], s8).wait()
                pltpu.make_async_copy(rr0, er_hbm.at[pl.ds(0, bg)], s9).wait()
                pltpu.make_async_copy(cr1, ec_hbm.at[pl.ds(0, bg)], s10).wait()
                pltpu.make_async_copy(rr1, er_hbm.at[pl.ds(0, bg)], s11).wait()

            di0c.wait()
            g0c = pltpu.async_copy(c_hbm.at[cg0], cr0, s4)
            di0r.wait()
            g0r = pltpu.async_copy(r_hbm.at[ri0], rr0, s5)
            di1c.wait()
            g1c = pltpu.async_copy(c_hbm.at[cg1], cr1, s6)
            di1r.wait()
            g1r = pltpu.async_copy(r_hbm.at[ri1], rr1, s7)
            g0c.wait()
            pltpu.async_copy(cr0, ec_hbm.at[pl.ds(off0, bg)], s8)
            g0r.wait()
            pltpu.async_copy(rr0, er_hbm.at[pl.ds(off0, bg)], s9)
            g1c.wait()
            pltpu.async_copy(cr1, ec_hbm.at[pl.ds(off1, bg)], s10)
            g1r.wait()
            pltpu.async_copy(rr1, er_hbm.at[pl.ds(off1, bg)], s11)

        pltpu.make_async_copy(cr0, ec_hbm.at[pl.ds(0, bg)], s8).wait()
        pltpu.make_async_copy(rr0, er_hbm.at[pl.ds(0, bg)], s9).wait()
        pltpu.make_async_copy(cr1, ec_hbm.at[pl.ds(0, bg)], s10).wait()
        pltpu.make_async_copy(rr1, er_hbm.at[pl.ds(0, bg)], s11).wait()

    return kg


def _make_scatter_kernel(n_acc, n_rng, rng):
    rpt = rng // _NS
    dp = 128

    @functools.partial(
        pl.kernel,
        out_type=jax.ShapeDtypeStruct((_NC, n_acc, dp), _F32),
        mesh=_vector_mesh(),
        scratch_types=[
            pltpu.VMEM((16,), _I32),
            pltpu.VMEM((_B,), _I32),
            pltpu.VMEM((_B,), _I32),
            pltpu.VMEM((_B, dp), _F32),
            pltpu.VMEM((_B, dp), _F32),
            pltpu.VMEM((rpt, dp), _F32),
            pltpu.VMEM_SHARED((rng, dp), _F32),
        ] + [pltpu.SemaphoreType.DMA] * 6,
    )
    def ks(msg_hbm, coll_hbm, goff_hbm, out_hbm,
           offs_v, clA, clB, rowsA, rowsB, zbuf, acc_sh,
           s0, s1, s2, s3, s4, s5):
        cid = lax.axis_index("c")
        sid = lax.axis_index("s")
        wid = sid * _NC + cid
        zero16 = jnp.zeros((_L,), _F32)

        pltpu.sync_copy(goff_hbm, offs_v)
        offs = offs_v[...]

        @pl.loop(0, rpt)
        def _(i):
            for kk in range(dp // _L):
                zbuf[i, pl.ds(kk * _L, _L)] = zero16

        for r in range(n_rng):
            pltpu.sync_copy(zbuf, acc_sh.at[pl.ds(sid * rpt, rpt)])
            plsc.subcore_barrier()
            b0 = offs[r] // _B
            b1 = offs[r + 1] // _B

            @pl.loop(b0 + wid, b1, step=2 * _NW)
            def _(b):
                offA = b * _B
                offB = (b + _NW) * _B
                hb = b + _NW < b1
                dAc = pltpu.async_copy(coll_hbm.at[pl.ds(offA, _B)], clA, s0)
                dAm = pltpu.async_copy(msg_hbm.at[pl.ds(offA, _B)], rowsA, s1)

                @pl.when(hb)
                def _():
                    pltpu.async_copy(coll_hbm.at[pl.ds(offB, _B)], clB, s2)
                    pltpu.async_copy(msg_hbm.at[pl.ds(offB, _B)], rowsB, s3)

                dAc.wait()
                dAm.wait()
                sA = pltpu.async_copy(rowsA, acc_sh.at[clA], s4, add=True)

                @pl.when(hb)
                def _():
                    pltpu.make_async_copy(
                        coll_hbm.at[pl.ds(offB, _B)], clB, s2).wait()
                    pltpu.make_async_copy(
                        msg_hbm.at[pl.ds(offB, _B)], rowsB, s3).wait()
                    pltpu.async_copy(rowsB, acc_sh.at[clB], s5, add=True)

                sA.wait()

                @pl.when(hb)
                def _():
                    pltpu.make_async_copy(rowsB, acc_sh.at[clB], s5).wait()

            plsc.subcore_barrier()
            pltpu.sync_copy(
                acc_sh.at[pl.ds(sid * rpt, rpt)],
                out_hbm.at[cid, pl.ds(r * rng + sid * rpt, rpt)])
            plsc.subcore_barrier()

    return ks


_BLKE = 2048


def _tc_edge_coeff(ec, er):
    ep = ec.shape[0]

    def body(ec_ref, er_ref, msg_ref):
        xc = _unpack_pairs(ec_ref[:, :64], _BLKE)
        tt = _unpack_pairs(ec_ref[:, 64:128], _BLKE)
        gg = _unpack_pairs(ec_ref[:, 128:192], _BLKE)
        xj = _unpack_pairs(er_ref[:, :64], _BLKE)
        xm = _unpack_pairs(er_ref[:, 64:128], _BLKE)
        df = xj - xc
        dd = df * df
        ss = jnp.sum(dd, axis=-1, keepdims=True)
        tn = jnp.sum(tt * dd, axis=-1, keepdims=True)
        u = xc * xj
        wi = jnp.sum(gg * u, axis=-1, keepdims=True)
        ni = jnp.sum(gg * xc * xc, axis=-1, keepdims=True)
        nj = jnp.sum(gg * xj * xj, axis=-1, keepdims=True)
        tau = tn / jnp.maximum(ss, 1e-16)
        den = jnp.sqrt(jnp.maximum(ni * nj, 1e-30)) + 1e-8
        alpha = jax.nn.sigmoid(wi / den)
        msg_ref[...] = tau * alpha * xm

    return pl.pallas_call(
        body,
        grid=(ep // _BLKE,),
        in_specs=[
            pl.BlockSpec((_BLKE, 256), lambda i: (i, 0)),
            pl.BlockSpec((_BLKE, 128), lambda i: (i, 0)),
        ],
        out_specs=pl.BlockSpec((_BLKE, 128), lambda i: (i, 0)),
        out_shape=jax.ShapeDtypeStruct((ep, 128), _F32),
    )(ec, er)


def _ln(h, g, b):
    mu = jnp.mean(h, axis=-1, keepdims=True)
    var = jnp.mean((h - mu) * (h - mu), axis=-1, keepdims=True)
    return (h - mu) / jnp.sqrt(var + 1e-5) * g + b


_BLK = 1024


def _tc_encoder(xp, p):
    n_pad = xp.shape[0]

    def body(x_ref, w_ref, b_ref, g_ref, bt_ref, o_ref):
        h = jnp.dot(x_ref[...], w_ref[...],
                    preferred_element_type=_F32) + b_ref[...]
        o_ref[...] = jnp.maximum(_ln(h, g_ref[...], bt_ref[...]), 0.0)

    d_in = xp.shape[1]
    hid = p['enc_W'].shape[1]
    return pl.pallas_call(
        body,
        grid=(n_pad // _BLK,),
        in_specs=[
            pl.BlockSpec((_BLK, d_in), lambda i: (i, 0)),
            pl.BlockSpec((d_in, hid), lambda i: (0, 0)),
            pl.BlockSpec((hid,), lambda i: (0,)),
            pl.BlockSpec((hid,), lambda i: (0,)),
            pl.BlockSpec((hid,), lambda i: (0,)),
        ],
        out_specs=pl.BlockSpec((_BLK, hid), lambda i: (i, 0)),
        out_shape=jax.ShapeDtypeStruct((n_pad, hid), _F32),
    )(xp, p['enc_W'], p['enc_b'], p['enc_g'], p['enc_bt'])


def _tc_dense(h, ns2, deg2, l, dout):
    n_pad = h.shape[0]
    rw = 256

    def body(h_ref, ns_ref, deg_ref, mw1_ref, mb1_ref, mg_ref, mbt_ref,
             mw2_ref, mb2_ref, msgw_ref, selfw_ref, selfb_ref,
             c_ref, r_ref, so_ref):
        hh = h_ref[...]
        s = ns_ref[0] + ns_ref[1] + hh
        cnt = deg_ref[...] + 1.0
        x_nb = s / jnp.maximum(cnt, 1.0)[:, None]
        comb = jnp.concatenate([hh, x_nb], axis=-1)
        hm = jnp.dot(comb, mw1_ref[...],
                     preferred_element_type=_F32) + mb1_ref[...]
        hm = jnp.maximum(_ln(hm, mg_ref[...], mbt_ref[...]), 0.0)
        raw = jnp.dot(hm, mw2_ref[...],
                      preferred_element_type=_F32) + mb2_ref[...]
        z2 = 2.0 * raw
        sp = jnp.maximum(z2, 0.0) + jnp.log1p(jnp.exp(-jnp.abs(z2)))
        g = jnp.clip(sp * 0.5, 0.001, 10.0)
        t = (1.0 - g * g) / (1.0 + g * g)
        xm = jnp.dot(hh, msgw_ref[...], preferred_element_type=_F32)
        so = jnp.dot(hh, selfw_ref[...],
                     preferred_element_type=_F32) + selfb_ref[...]
        hp = _pack_pairs(hh, _BLK)
        tp = _pack_pairs(t, _BLK)
        gp = _pack_pairs(g, _BLK)
        zp = jnp.zeros((_BLK, 64), _F32)
        if dout == 128:
            xmp = _pack_pairs(xm, _BLK)
        else:
            xmp = _pack_pairs(
                jnp.concatenate([xm, jnp.zeros((_BLK, 128 - dout), _F32)],
                                axis=-1), _BLK)
        c_ref[...] = jnp.concatenate([hp, tp, gp, zp], axis=-1)
        r_ref[...] = jnp.concatenate([hp, xmp], axis=-1)
        so_ref[...] = so

    return pl.pallas_call(
        body,
        grid=(n_pad // _BLK,),
        in_specs=[
            pl.BlockSpec((_BLK, 128), lambda i: (i, 0)),
            pl.BlockSpec((2, _BLK, 128), lambda i: (0, i, 0)),
            pl.BlockSpec((_BLK,), lambda i: (i,)),
            pl.BlockSpec((256, 64), lambda i: (0, 0)),
            pl.BlockSpec((64,), lambda i: (0,)),
            pl.BlockSpec((64,), lambda i: (0,)),
            pl.BlockSpec((64,), lambda i: (0,)),
            pl.BlockSpec((64, 128), lambda i: (0, 0)),
            pl.BlockSpec((128,), lambda i: (0,)),
            pl.BlockSpec((128, dout), lambda i: (0, 0)),
            pl.BlockSpec((128, dout), lambda i: (0, 0)),
            pl.BlockSpec((dout,), lambda i: (0,)),
        ],
        out_specs=[
            pl.BlockSpec((_BLK, 256), lambda i: (i, 0)),
            pl.BlockSpec((_BLK, 128), lambda i: (i, 0)),
            pl.BlockSpec((_BLK, dout), lambda i: (i, 0)),
        ],
        out_shape=[
            jax.ShapeDtypeStruct((n_pad, 256), _F32),
            jax.ShapeDtypeStruct((n_pad, 128), _F32),
            jax.ShapeDtypeStruct((n_pad, dout), _F32),
        ],
    )(h, ns2, deg2, l['mW1'], l['mb1'], l['mg'], l['mbt'],
      l['mW2'], l['mb2'], l['msgW'], l['selfW'], l['selfb'])


def _tc_epilogue(agg2, so, l, last):
    n_pad = so.shape[0]
    dout = so.shape[1]

    def body(agg_ref, so_ref, g_ref, bt_ref, o_ref):
        agg = agg_ref[0][:, :dout] + agg_ref[1][:, :dout]
        o = _ln(agg + so_ref[...], g_ref[...], bt_ref[...])
        if last:
            m = jnp.max(o, axis=-1, keepdims=True)
            lse = jnp.log(jnp.sum(jnp.exp(o - m), axis=-1, keepdims=True)) + m
            o_ref[...] = o - lse
        else:
            o_ref[...] = jnp.maximum(o, 0.0)

    return pl.pallas_call(
        body,
        grid=(n_pad // _BLK,),
        in_specs=[
            pl.BlockSpec((2, _BLK, 128), lambda i: (0, i, 0)),
            pl.BlockSpec((_BLK, dout), lambda i: (i, 0)),
            pl.BlockSpec((dout,), lambda i: (0,)),
            pl.BlockSpec((dout,), lambda i: (0,)),
        ],
        out_specs=pl.BlockSpec((_BLK, dout), lambda i: (i, 0)),
        out_shape=jax.ShapeDtypeStruct((n_pad, dout), _F32),
    )(agg2, so, l['ng'], l['nbt'])


def kernel(x, params, edge_index):
    n, d_in = x.shape
    e = edge_index.shape[1]

    n_rng = 8
    rng = _NS * 80
    n_acc = n_rng * rng
    row = edge_index[0]
    col = edge_index[1]

    grp = col // rng
    pos = jnp.zeros((e,), _I32)
    goffs = []
    base = jnp.int32(0)
    pad_counts = []
    for r in range(n_rng):
        m = grp == r
        ranks = jnp.cumsum(m.astype(_I32))
        size = ranks[-1]
        goffs.append(base)
        pos = jnp.where(m, base + ranks - 1, pos)
        padded = ((size + _B - 1) // _B) * _B
        pad_counts.append(padded - size)
        base = base + padded
    goffs.append(base)
    bpt = (e + n_rng * _B + _NW * _B - 1) // (_NW * _B)
    ep = bpt * _NW * _B
    rowp = jnp.full((ep,), n, _I32).at[pos].set(row)
    colp = jnp.zeros((ep,), _I32).at[pos].set(col - grp * rng)
    colg = jnp.full((ep,), n, _I32).at[pos].set(col)
    goff = jnp.zeros((16,), _I32).at[:n_rng + 1].set(jnp.stack(goffs))
    padvec = jnp.zeros((n_acc,), _F32).at[
        jnp.arange(n_rng) * rng].set(jnp.stack(pad_counts).astype(_F32))

    xp = jnp.pad(x, ((0, n_acc - n), (0, 0)))

    deg_out = _make_deg_kernel(n_acc, n_rng, rng)(colp, goff)
    deg = deg_out[0, :, 0] + deg_out[1, :, 0] - padvec
    h = _tc_encoder(xp, params)

    nsum = _make_nsum_kernel(n_acc, 128, n_rng, rng)
    gatherk = _make_gather_kernel(ep)
    scatterk = _make_scatter_kernel(n_acc, n_rng, rng)
    n_layers = len(params['layers'])
    for i, l in enumerate(params['layers']):
        dout = l['msgW'].shape[1]
        ns2 = nsum(h, rowp, colp, goff)
        c_tbl, r_tbl, so = _tc_dense(h, ns2, deg, l, dout)
        ec, er = gatherk(c_tbl, r_tbl, rowp, colg)
        msg = _tc_edge_coeff(ec, er)
        agg2 = scatterk(msg, colp, goff)
        h = _tc_epilogue(agg2, so, l, last=(i == n_layers - 1))

    return h[:n]
